# Initial kernel scaffold; baseline (speedup 1.0000x reference)
import jax
import jax.numpy as jnp
from jax.experimental import pallas as pl


def kernel(lit_idx, clause_idx, L_init_W, L_init_b, C_init_W, C_init_b, Lp_W, Lp_b, Lm_W1, Lm_b1, Lm_W2, Lm_b2, Cm_W1, Cm_b1, Cm_W2, Cm_b2, Cp_W1, Cp_b1, Cp_W2, Cp_b2, in_w1, in_b1, in_w2, in_b2, Lv_W1, Lv_b1, Lv_W2, Lv_b2):
    raise NotImplementedError("write your pallas kernel here")


# speedup vs baseline: 6.1988x; 6.1988x over previous
import functools

import jax
import jax.numpy as jnp
from jax import lax
from jax.experimental import pallas as pl
from jax.experimental.pallas import tpu as pltpu
from jax.experimental.pallas import tpu_sc as plsc

N_VARS = 50000
N_LITS = 100000
N_CLAUSES = 50000
N_CELLS = 800000
N_PROBS = 10
DIM = 64
R = 4
VARS_PER = N_VARS // N_PROBS
CLAUSES_PER = N_CLAUSES // N_PROBS

EROWS = N_CELLS // 128
EPS = 1e-6

_mesh = plsc.VectorSubcoreMesh(core_axis_name="c", subcore_axis_name="s")


def _zero_vmem_1d(ref, n):
    z = jnp.zeros((16,), jnp.float32)

    def body(i, _):
        ref[pl.ds(i * 16, 16)] = z
        return 0

    lax.fori_loop(0, n // 16, body, 0)


def _fill_vmem_1d(ref, n, value):
    v = jnp.full((16,), value, jnp.float32)

    def body(i, _):
        ref[pl.ds(i * 16, 16)] = v
        return 0

    lax.fori_loop(0, n // 16, body, 0)


def _spread_1d(sid, total, bs, fn):
    full = total // bs
    tail = total - full * bs
    rounds, rem = divmod(full, 16)
    for r in range(rounds):
        fn((sid + 16 * r) * bs, bs)
    if rem:
        @pl.when(sid < rem)
        def _():
            fn((16 * rounds + sid) * bs, bs)
    if tail:
        @pl.when(sid == 15)
        def _():
            fn(full * bs, tail)


def _edge_rows(sid, fn):
    def body(k, _):
        fn(sid + 16 * k)
        return 0

    lax.fori_loop(0, EROWS // 16, body, 0)
    rem = EROWS % 16
    if rem:
        @pl.when(sid < rem)
        def _():
            fn((EROWS // 16) * 16 + sid)


@functools.partial(
    pl.kernel,
    out_type=(
        jax.ShapeDtypeStruct((N_LITS,), jnp.float32),
        jax.ShapeDtypeStruct((N_CLAUSES,), jnp.float32),
        jax.ShapeDtypeStruct((N_CLAUSES,), jnp.float32),
        jax.ShapeDtypeStruct((N_CLAUSES,), jnp.float32),
    ),
    mesh=_mesh,
    scratch_types=[
        pltpu.VMEM_SHARED((N_LITS,), jnp.float32),
        pltpu.VMEM_SHARED((N_CLAUSES,), jnp.float32),
        pltpu.VMEM_SHARED((N_CLAUSES,), jnp.float32),
        pltpu.VMEM((2048,), jnp.float32),
        pltpu.VMEM((2000,), jnp.float32),
        pltpu.VMEM((128,), jnp.float32),
        pltpu.VMEM((128,), jnp.int32),
        pltpu.VMEM((128,), jnp.int32),
        pltpu.VMEM((128,), jnp.float32),
    ],
)
def _k1(lit2d, cls2d, deg_lit_out, deg_cl_out, t0_out, t1_out,
        deg_lit_sh, deg_cl_sh, t_sh, zbuf, stage, ones_v, lit_v, cls_v, val_v):
    cid = lax.axis_index("c")
    sid = lax.axis_index("s")

    _zero_vmem_1d(zbuf, 2048)
    _fill_vmem_1d(ones_v, 128, 1.0)

    def zero_to(sh):
        def fn(off, size):
            pltpu.sync_copy(zbuf.at[pl.ds(0, size)], sh.at[pl.ds(off, size)])
        return fn

    _spread_1d(sid, N_LITS, 2000, zero_to(deg_lit_sh))
    _spread_1d(sid, N_CLAUSES, 2000, zero_to(deg_cl_sh))
    _spread_1d(sid, N_CLAUSES, 2000, zero_to(t_sh))
    plsc.subcore_barrier()

    def hist_row(j):
        pltpu.sync_copy(lit2d.at[j], lit_v)
        pltpu.sync_copy(cls2d.at[j], cls_v)
        pltpu.sync_copy(ones_v, deg_lit_sh.at[lit_v], add=True)
        pltpu.sync_copy(ones_v, deg_cl_sh.at[cls_v], add=True)

    _edge_rows(sid, hist_row)
    plsc.subcore_barrier()

    half = EROWS // 2
    base = cid * half

    def t_row(j):
        pltpu.sync_copy(lit2d.at[j], lit_v)
        pltpu.sync_copy(cls2d.at[j], cls_v)
        pltpu.sync_copy(deg_lit_sh.at[lit_v], val_v)
        pltpu.sync_copy(val_v, t_sh.at[cls_v], add=True)

    def t_row_k(k, _):
        t_row(base + sid + 16 * k)
        return 0

    lax.fori_loop(0, half // 16, t_row_k, 0)
    rem = half % 16
    if rem:
        @pl.when(sid < rem)
        def _():
            t_row(base + (half // 16) * 16 + sid)
    plsc.subcore_barrier()

    def publish(sh, out):
        def fn(off, size):
            pltpu.sync_copy(sh.at[pl.ds(off, size)], stage.at[pl.ds(0, size)])
            pltpu.sync_copy(stage.at[pl.ds(0, size)], out.at[pl.ds(off, size)])
        return fn

    @pl.when(cid == 0)
    def _():
        _spread_1d(sid, N_LITS, 2000, publish(deg_lit_sh, deg_lit_out))
        _spread_1d(sid, N_CLAUSES, 2000, publish(deg_cl_sh, deg_cl_out))
        _spread_1d(sid, N_CLAUSES, 2000, publish(t_sh, t0_out))

    @pl.when(cid == 1)
    def _():
        _spread_1d(sid, N_CLAUSES, 2000, publish(t_sh, t1_out))


def _k3_body(ta_ref, tb_ref, d1_ref, LpW_ref, Lpb_ref, CmW1_ref, Cmb1_ref,
             CmW2_ref, Cmb2_ref, CpW1_ref, Cpb1_ref, CpW2_ref, Cpb2_ref,
             LiW_ref, Lib_ref, CiW_ref, Cib_ref, iw1_ref, ib1_ref, out_ref):
    n = float(CLAUSES_PER)
    t_col = ta_ref[...] + tb_ref[...]
    d_col = d1_ref[...]
    St = jnp.sum(t_col) / n
    Sd = jnp.sum(d_col) / n
    Vt = jnp.sum(t_col * t_col) / n - St * St
    Vd = jnp.sum(d_col * d_col) / n - Sd * Sd
    Ctd = jnp.sum(t_col * d_col) / n - St * Sd
    w = LiW_ref[...]
    b = Lib_ref[...]
    iw1 = iw1_ref[...]
    ib1 = ib1_ref[...]
    acc = d_col * CiW_ref[...] + Cib_ref[...]
    for i in range(R):
        Wp = LpW_ref[i]
        u = jnp.dot(w, Wp, preferred_element_type=jnp.float32)
        v = jnp.dot(b, Wp, preferred_element_type=jnp.float32) + Lpb_ref[i][None]
        var = Vt * u * u + 2.0 * Ctd * u * v + Vd * v * v
        std = jnp.sqrt(var + EPS)
        ai = iw1 * u / std
        bi = iw1 * v / std
        ei = ib1 - (St * u + Sd * v) * iw1 / std
        A = jnp.dot(ai, CmW1_ref[i], preferred_element_type=jnp.float32)
        B = jnp.dot(bi, CmW1_ref[i], preferred_element_type=jnp.float32)
        E = (jnp.dot(ei, CmW1_ref[i], preferred_element_type=jnp.float32)
             + Cmb1_ref[i][None])
        H = jnp.maximum(t_col * A + d_col * B + E, 0.0)
        acc = (acc + jnp.dot(H, CmW2_ref[i], preferred_element_type=jnp.float32)
               + Cmb2_ref[i][None])
    z = jnp.maximum(
        jnp.dot(acc, CpW1_ref[R - 1], preferred_element_type=jnp.float32)
        + Cpb1_ref[R - 1][None], 0.0)
    out_ref[...] = (jnp.dot(z, CpW2_ref[R - 1], preferred_element_type=jnp.float32)
                    + Cpb2_ref[R - 1][None])


def _full(shape):
    return pl.BlockSpec(shape, lambda *args: tuple(0 for _ in shape))


def _k3(ta, tb, d1, LpW, Lpb, CmW1, Cmb1, CmW2, Cmb2, CpW1, Cpb1, CpW2, Cpb2,
        LiW, Lib, CiW, Cib, iw1, ib1):
    return pl.pallas_call(
        _k3_body,
        grid=(N_PROBS,),
        in_specs=[
            pl.BlockSpec((CLAUSES_PER, 1), lambda p: (p, 0)),
            pl.BlockSpec((CLAUSES_PER, 1), lambda p: (p, 0)),
            pl.BlockSpec((CLAUSES_PER, 1), lambda p: (p, 0)),
            _full((R, DIM, DIM)), _full((R, DIM)),
            _full((R, DIM, DIM)), _full((R, DIM)),
            _full((R, DIM, DIM)), _full((R, DIM)),
            _full((R, DIM, DIM)), _full((R, DIM)),
            _full((R, DIM, DIM)), _full((R, DIM)),
            _full((1, DIM)), _full((1, DIM)), _full((1, DIM)), _full((1, DIM)),
            _full((1, DIM)), _full((1, DIM)),
        ],
        out_specs=pl.BlockSpec((CLAUSES_PER, DIM), lambda p: (p, 0)),
        out_shape=jax.ShapeDtypeStruct((N_CLAUSES, DIM), jnp.float32),
    )(ta, tb, d1, LpW, Lpb, CmW1, Cmb1, CmW2, Cmb2, CpW1, Cpb1, CpW2, Cpb2,
      LiW, Lib, CiW, Cib, iw1, ib1)


CHUNK_LITS = 25000
ACC_ROWS = CHUNK_LITS


@functools.partial(
    pl.kernel,
    out_type=jax.ShapeDtypeStruct((N_LITS, DIM), jnp.float32),
    mesh=_mesh,
    scratch_types=[
        pltpu.VMEM_SHARED((ACC_ROWS, DIM), jnp.float32),
        pltpu.VMEM((32, DIM), jnp.float32),
        pltpu.VMEM((128,), jnp.int32),
        pltpu.VMEM((128,), jnp.int32),
        pltpu.VMEM((128,), jnp.int32),
        pltpu.VMEM((128,), jnp.int32),
        pltpu.VMEM((128, DIM), jnp.float32),
        pltpu.VMEM((200, DIM), jnp.float32),
    ],
    compiler_params=pltpu.CompilerParams(use_tc_tiling_on_sc=False),
)
def _k4(cp_hbm, lit2d, cls2d, cl_out,
        acc_sh, zrows, lit_v, cls_v, gidx_v, off_v, rows_v, stage_rows):
    cid = lax.axis_index("c")
    sid = lax.axis_index("s")

    def zrow_body(i, _):
        z = jnp.zeros((16,), jnp.float32)
        for q in range(DIM // 16):
            zrows[i, pl.ds(q * 16, 16)] = z
        return 0

    lax.fori_loop(0, 32, zrow_body, 0)

    for kchunk in range(2):
        chunk = 2 * cid + kchunk
        lo = chunk * CHUNK_LITS

        full_rounds, rem = divmod(ACC_ROWS // 32, 16)
        for r in range(full_rounds):
            b = sid + 16 * r
            pltpu.sync_copy(zrows, acc_sh.at[pl.ds(b * 32, 32), :])
        if rem:
            @pl.when(sid < rem)
            def _():
                b = full_rounds * 16 + sid
                pltpu.sync_copy(zrows, acc_sh.at[pl.ds(b * 32, 32), :])
        tail_rows = ACC_ROWS - (ACC_ROWS // 32) * 32
        if tail_rows:
            @pl.when(sid == 15)
            def _():
                pltpu.sync_copy(
                    zrows.at[pl.ds(0, tail_rows), :],
                    acc_sh.at[pl.ds((ACC_ROWS // 32) * 32, tail_rows), :])
        plsc.subcore_barrier()

        def scan_row(j):
            pltpu.sync_copy(lit2d.at[j], lit_v)
            pltpu.sync_copy(cls2d.at[j], cls_v)
            for q in range(8):
                l16 = lit_v[pl.ds(q * 16, 16)]
                c16 = cls_v[pl.ds(q * 16, 16)]
                inm = (l16 >= lo) & (l16 < lo + CHUNK_LITS)
                m1 = jnp.full((16,), -1, jnp.int32)
                gidx_v[pl.ds(q * 16, 16)] = jnp.where(inm, c16, m1)
                off_v[pl.ds(q * 16, 16)] = jnp.where(inm, l16 - lo, m1)
            pltpu.sync_copy(
                cp_hbm.at[plsc.Indices(gidx_v, ignored_value=-1)],
                rows_v)
            pltpu.sync_copy(
                rows_v, acc_sh.at[plsc.Indices(off_v, ignored_value=-1)],
                add=True)

        _edge_rows(sid, scan_row)
        plsc.subcore_barrier()

        def wb(b):
            r0 = b * 200
            pltpu.sync_copy(acc_sh.at[pl.ds(r0, 200), :], stage_rows)
            pltpu.sync_copy(stage_rows, cl_out.at[pl.ds(lo + r0, 200), :])

        def wb_round(r, _):
            wb(sid + 16 * r)
            return 0

        lax.fori_loop(0, 7, wb_round, 0)
        @pl.when(sid < 13)
        def _():
            wb(112 + sid)
        plsc.subcore_barrier()


def _k5_body(cl_ref, degf_ref, degs_ref, LiW_ref, Lib_ref,
             sums_ref, sumsq_ref, sumdeg_ref):
    p = pl.program_id(0)
    h = pl.program_id(1)
    L = cl_ref[...] + degf_ref[...] * LiW_ref[...] + Lib_ref[...]
    s1 = jnp.sum(L, axis=0, keepdims=True)
    s2 = jnp.sum(L * L, axis=0, keepdims=True)
    sd = jnp.sum(degs_ref[...]) * jnp.ones((1, DIM), jnp.float32)

    @pl.when(h == 0)
    def _():
        sums_ref[pl.ds(p, 1), :] = s1
        sumsq_ref[pl.ds(p, 1), :] = s2
        sumdeg_ref[pl.ds(p, 1), :] = sd

    @pl.when(h == 1)
    def _():
        sums_ref[pl.ds(p, 1), :] += s1
        sumsq_ref[pl.ds(p, 1), :] += s2
        sumdeg_ref[pl.ds(p, 1), :] += sd


def _k5(cl, deg1, LiW, Lib):
    return pl.pallas_call(
        _k5_body,
        grid=(N_PROBS, 2),
        in_specs=[
            pl.BlockSpec((VARS_PER, DIM), lambda p, h: (p + N_PROBS * h, 0)),
            pl.BlockSpec((VARS_PER, 1), lambda p, h: (p + N_PROBS * (1 - h), 0)),
            pl.BlockSpec((VARS_PER, 1), lambda p, h: (p + N_PROBS * h, 0)),
            _full((1, DIM)), _full((1, DIM)),
        ],
        out_specs=[
            _full((N_PROBS, DIM)),
            _full((N_PROBS, DIM)),
            _full((N_PROBS, DIM)),
        ],
        out_shape=[
            jax.ShapeDtypeStruct((N_PROBS, DIM), jnp.float32),
            jax.ShapeDtypeStruct((N_PROBS, DIM), jnp.float32),
            jax.ShapeDtypeStruct((N_PROBS, DIM), jnp.float32),
        ],
    )(cl, deg1, deg1, LiW, Lib)


def _k6_body(cl_ref, degf_ref, LiW_ref, Lib_ref, sums_ref, sumsq_ref,
             sumdeg_ref, iw2_ref, ib2_ref, LmW1_ref, Lmb1_ref, LmW2_ref,
             Lmb2_ref, LvW1_ref, Lvb1_ref, LvW2_ref, Lvb2_ref,
             out_ref, hacc_ref):
    p = pl.program_id(0)
    h = pl.program_id(1)
    ntot = float(2 * VARS_PER)
    mean = sums_ref[pl.ds(p, 1), :] / ntot
    var = sumsq_ref[pl.ds(p, 1), :] / ntot - mean * mean
    std = jnp.sqrt(var + EPS)
    w = LiW_ref[...]
    b = Lib_ref[...]
    L = cl_ref[...] + degf_ref[...] * w + b
    Ln = iw2_ref[...] * (L - mean) / std + ib2_ref[...]
    Hh = jnp.maximum(
        jnp.dot(Ln, LmW1_ref[R - 1], preferred_element_type=jnp.float32)
        + Lmb1_ref[R - 1][None], 0.0)
    hsum = jnp.sum(Hh, axis=0, keepdims=True)

    @pl.when(h == 0)
    def _():
        hacc_ref[...] = hsum

    @pl.when(h == 1)
    def _():
        Hbar = (hacc_ref[...] + hsum) / ntot
        rep = (jnp.dot(Hbar, LmW2_ref[R - 1], preferred_element_type=jnp.float32)
               + Lmb2_ref[R - 1][None]
               + (sumdeg_ref[pl.ds(p, 1), :] / ntot) * w + b)
        z = jnp.maximum(
            jnp.dot(rep, LvW1_ref[...], preferred_element_type=jnp.float32)
            + Lvb1_ref[...], 0.0)
        out_ref[pl.ds(p, 1), :] = (jnp.dot(z, LvW2_ref[...],
                                           preferred_element_type=jnp.float32)
                                   + Lvb2_ref[...])


def _k6(cl, deg1, LiW, Lib, sums, sumsq, sumdeg, iw2, ib2,
        LmW1, Lmb1, LmW2, Lmb2, LvW1, Lvb1, LvW2, Lvb2):
    return pl.pallas_call(
        _k6_body,
        grid=(N_PROBS, 2),
        in_specs=[
            pl.BlockSpec((VARS_PER, DIM), lambda p, h: (p + N_PROBS * h, 0)),
            pl.BlockSpec((VARS_PER, 1), lambda p, h: (p + N_PROBS * (1 - h), 0)),
            _full((1, DIM)), _full((1, DIM)),
            _full((N_PROBS, DIM)),
            _full((N_PROBS, DIM)),
            _full((N_PROBS, DIM)),
            _full((1, DIM)), _full((1, DIM)),
            _full((R, DIM, DIM)), _full((R, DIM)),
            _full((R, DIM, DIM)), _full((R, DIM)),
            _full((DIM, DIM)), _full((1, DIM)),
            _full((DIM, DIM)), _full((1, DIM)),
        ],
        out_specs=_full((N_PROBS, DIM)),
        out_shape=jax.ShapeDtypeStruct((N_PROBS, DIM), jnp.float32),
        scratch_shapes=[pltpu.VMEM((1, DIM), jnp.float32)],
    )(cl, deg1, LiW, Lib, sums, sumsq, sumdeg, iw2, ib2,
      LmW1, Lmb1, LmW2, Lmb2, LvW1, Lvb1, LvW2, Lvb2)


def kernel(lit_idx, clause_idx, L_init_W, L_init_b, C_init_W, C_init_b,
           Lp_W, Lp_b, Lm_W1, Lm_b1, Lm_W2, Lm_b2, Cm_W1, Cm_b1, Cm_W2,
           Cm_b2, Cp_W1, Cp_b1, Cp_W2, Cp_b2, in_w1, in_b1, in_w2, in_b2,
           Lv_W1, Lv_b1, Lv_W2, Lv_b2):
    lit2d = lit_idx.reshape(EROWS, 128)
    cls2d = clause_idx.reshape(EROWS, 128)

    deg_lit, deg_clause, t0, t1 = _k1(lit2d, cls2d)

    ta = t0.reshape(N_CLAUSES, 1)
    tb = t1.reshape(N_CLAUSES, 1)
    d1 = deg_clause.reshape(N_CLAUSES, 1)
    Lib = L_init_b.reshape(1, DIM)
    Cib = C_init_b.reshape(1, DIM)
    iw1 = in_w1.reshape(1, DIM)
    ib1 = in_b1.reshape(1, DIM)
    cp = _k3(ta, tb, d1, Lp_W, Lp_b, Cm_W1, Cm_b1, Cm_W2, Cm_b2,
             Cp_W1, Cp_b1, Cp_W2, Cp_b2, L_init_W, Lib, C_init_W, Cib,
             iw1, ib1)

    cl = _k4(cp, lit2d, cls2d)

    deg1 = deg_lit.reshape(N_LITS, 1)
    sums, sumsq, sumdeg = _k5(cl, deg1, L_init_W, Lib)
    out = _k6(cl, deg1, L_init_W, Lib, sums, sumsq, sumdeg,
              in_w2.reshape(1, DIM), in_b2.reshape(1, DIM),
              Lm_W1, Lm_b1, Lm_W2, Lm_b2,
              Lv_W1, Lv_b1.reshape(1, DIM), Lv_W2, Lv_b2.reshape(1, DIM))
    return out


# speedup vs baseline: 11.9401x; 1.9262x over previous
import functools

import jax
import jax.numpy as jnp
from jax import lax
from jax.experimental import pallas as pl
from jax.experimental.pallas import tpu as pltpu
from jax.experimental.pallas import tpu_sc as plsc

N_VARS = 50000
N_LITS = 100000
N_CLAUSES = 50000
N_CELLS = 800000
N_PROBS = 10
DIM = 64
R = 4
VARS_PER = N_VARS // N_PROBS
CLAUSES_PER = N_CLAUSES // N_PROBS

EROWS = N_CELLS // 128
EPS = 1e-6

_mesh = plsc.VectorSubcoreMesh(core_axis_name="c", subcore_axis_name="s")


def _zero_vmem_1d(ref, n):
    z = jnp.zeros((16,), jnp.float32)

    def body(i, _):
        ref[pl.ds(i * 16, 16)] = z
        return 0

    lax.fori_loop(0, n // 16, body, 0)


def _fill_vmem_1d(ref, n, value):
    v = jnp.full((16,), value, jnp.float32)

    def body(i, _):
        ref[pl.ds(i * 16, 16)] = v
        return 0

    lax.fori_loop(0, n // 16, body, 0)


def _spread_1d(sid, total, bs, fn):
    full = total // bs
    tail = total - full * bs
    rounds, rem = divmod(full, 16)
    for r in range(rounds):
        fn((sid + 16 * r) * bs, bs)
    if rem:
        @pl.when(sid < rem)
        def _():
            fn((16 * rounds + sid) * bs, bs)
    if tail:
        @pl.when(sid == 15)
        def _():
            fn(full * bs, tail)


def _edge_rows(sid, fn):
    def body(k, _):
        fn(sid + 16 * k)
        return 0

    lax.fori_loop(0, EROWS // 16, body, 0)
    rem = EROWS % 16
    if rem:
        @pl.when(sid < rem)
        def _():
            fn((EROWS // 16) * 16 + sid)


@functools.partial(
    pl.kernel,
    out_type=(
        jax.ShapeDtypeStruct((N_LITS,), jnp.float32),
        jax.ShapeDtypeStruct((N_CLAUSES,), jnp.float32),
        jax.ShapeDtypeStruct((N_CLAUSES,), jnp.float32),
        jax.ShapeDtypeStruct((N_CLAUSES,), jnp.float32),
    ),
    mesh=_mesh,
    scratch_types=[
        pltpu.VMEM_SHARED((N_LITS,), jnp.float32),
        pltpu.VMEM_SHARED((N_CLAUSES,), jnp.float32),
        pltpu.VMEM_SHARED((N_CLAUSES,), jnp.float32),
        pltpu.VMEM((2048,), jnp.float32),
        pltpu.VMEM((2000,), jnp.float32),
        pltpu.VMEM((128,), jnp.float32),
        pltpu.VMEM((5, 128), jnp.int32),
        pltpu.VMEM((5, 128), jnp.int32),
        pltpu.VMEM((5, 128), jnp.float32),
        pltpu.SemaphoreType.DMA,
        pltpu.SemaphoreType.DMA,
    ],
    compiler_params=pltpu.CompilerParams(use_tc_tiling_on_sc=False),
)
def _k1(lit2d, cls2d, deg_lit_out, deg_cl_out, t0_out, t1_out,
        deg_lit_sh, deg_cl_sh, t_sh, zbuf, stage, ones_v, litb, clsb, valb,
        semA, semB):
    cid = lax.axis_index("c")
    sid = lax.axis_index("s")

    _zero_vmem_1d(zbuf, 2048)
    _fill_vmem_1d(ones_v, 128, 1.0)

    def zero_to(sh):
        def fn(off, size):
            pltpu.sync_copy(zbuf.at[pl.ds(0, size)], sh.at[pl.ds(off, size)])
        return fn

    _spread_1d(sid, N_LITS, 2000, zero_to(deg_lit_sh))
    _spread_1d(sid, N_CLAUSES, 2000, zero_to(deg_cl_sh))
    _spread_1d(sid, N_CLAUSES, 2000, zero_to(t_sh))
    plsc.subcore_barrier()

    def hist_batch(b, _):
        r0 = sid * 390 + 5 * b
        pltpu.sync_copy(lit2d.at[pl.ds(r0, 5), :], litb)
        pltpu.sync_copy(cls2d.at[pl.ds(r0, 5), :], clsb)
        ds = []
        for i in range(5):
            ds.append(pltpu.async_copy(
                ones_v, deg_lit_sh.at[litb.at[i]], semA, add=True))
            ds.append(pltpu.async_copy(
                ones_v, deg_cl_sh.at[clsb.at[i]], semB, add=True))
        for d in ds:
            d.wait()
        return 0

    lax.fori_loop(0, 78, hist_batch, 0)
    @pl.when(sid < EROWS % 16)
    def _():
        j = (EROWS // 16) * 16 + sid
        pltpu.sync_copy(lit2d.at[pl.ds(j, 1), :], litb.at[pl.ds(0, 1), :])
        pltpu.sync_copy(cls2d.at[pl.ds(j, 1), :], clsb.at[pl.ds(0, 1), :])
        pltpu.sync_copy(ones_v, deg_lit_sh.at[litb.at[0]], add=True)
        pltpu.sync_copy(ones_v, deg_cl_sh.at[clsb.at[0]], add=True)
    plsc.subcore_barrier()

    base = cid * (EROWS // 2) + sid * 195

    def t_batch(b, _):
        r0 = base + 5 * b
        pltpu.sync_copy(lit2d.at[pl.ds(r0, 5), :], litb)
        pltpu.sync_copy(cls2d.at[pl.ds(r0, 5), :], clsb)
        gs = [pltpu.async_copy(deg_lit_sh.at[litb.at[i]], valb.at[i], semA)
              for i in range(5)]
        for g in gs:
            g.wait()
        ss = [pltpu.async_copy(valb.at[i], t_sh.at[clsb.at[i]], semB, add=True)
              for i in range(5)]
        for s in ss:
            s.wait()
        return 0

    lax.fori_loop(0, 39, t_batch, 0)
    @pl.when(sid < 5)
    def _():
        j = cid * (EROWS // 2) + 3120 + sid
        pltpu.sync_copy(lit2d.at[pl.ds(j, 1), :], litb.at[pl.ds(0, 1), :])
        pltpu.sync_copy(cls2d.at[pl.ds(j, 1), :], clsb.at[pl.ds(0, 1), :])
        pltpu.sync_copy(deg_lit_sh.at[litb.at[0]], valb.at[0])
        pltpu.sync_copy(valb.at[0], t_sh.at[clsb.at[0]], add=True)
    plsc.subcore_barrier()

    def publish(sh, out):
        def fn(off, size):
            pltpu.sync_copy(sh.at[pl.ds(off, size)], stage.at[pl.ds(0, size)])
            pltpu.sync_copy(stage.at[pl.ds(0, size)], out.at[pl.ds(off, size)])
        return fn

    @pl.when(cid == 0)
    def _():
        _spread_1d(sid, N_LITS, 2000, publish(deg_lit_sh, deg_lit_out))
        _spread_1d(sid, N_CLAUSES, 2000, publish(deg_cl_sh, deg_cl_out))
        _spread_1d(sid, N_CLAUSES, 2000, publish(t_sh, t0_out))

    @pl.when(cid == 1)
    def _():
        _spread_1d(sid, N_CLAUSES, 2000, publish(t_sh, t1_out))


def _k3_body(ta_ref, tb_ref, d1_ref, LpW_ref, Lpb_ref, CmW1_ref, Cmb1_ref,
             CmW2_ref, Cmb2_ref, CpW1_ref, Cpb1_ref, CpW2_ref, Cpb2_ref,
             LiW_ref, Lib_ref, CiW_ref, Cib_ref, iw1_ref, ib1_ref, out_ref):
    n = float(CLAUSES_PER)
    t_col = ta_ref[...] + tb_ref[...]
    d_col = d1_ref[...]
    St = jnp.sum(t_col) / n
    Sd = jnp.sum(d_col) / n
    Vt = jnp.sum(t_col * t_col) / n - St * St
    Vd = jnp.sum(d_col * d_col) / n - Sd * Sd
    Ctd = jnp.sum(t_col * d_col) / n - St * Sd
    w = LiW_ref[...]
    b = Lib_ref[...]
    iw1 = iw1_ref[...]
    ib1 = ib1_ref[...]
    acc = d_col * CiW_ref[...] + Cib_ref[...]
    for i in range(R):
        Wp = LpW_ref[i]
        u = jnp.dot(w, Wp, preferred_element_type=jnp.float32)
        v = jnp.dot(b, Wp, preferred_element_type=jnp.float32) + Lpb_ref[i][None]
        var = Vt * u * u + 2.0 * Ctd * u * v + Vd * v * v
        std = jnp.sqrt(var + EPS)
        ai = iw1 * u / std
        bi = iw1 * v / std
        ei = ib1 - (St * u + Sd * v) * iw1 / std
        A = jnp.dot(ai, CmW1_ref[i], preferred_element_type=jnp.float32)
        B = jnp.dot(bi, CmW1_ref[i], preferred_element_type=jnp.float32)
        E = (jnp.dot(ei, CmW1_ref[i], preferred_element_type=jnp.float32)
             + Cmb1_ref[i][None])
        H = jnp.maximum(t_col * A + d_col * B + E, 0.0)
        acc = (acc + jnp.dot(H, CmW2_ref[i], preferred_element_type=jnp.float32)
               + Cmb2_ref[i][None])
    z = jnp.maximum(
        jnp.dot(acc, CpW1_ref[R - 1], preferred_element_type=jnp.float32)
        + Cpb1_ref[R - 1][None], 0.0)
    out_ref[...] = (jnp.dot(z, CpW2_ref[R - 1], preferred_element_type=jnp.float32)
                    + Cpb2_ref[R - 1][None])


def _full(shape):
    return pl.BlockSpec(shape, lambda *args: tuple(0 for _ in shape))


def _k3(ta, tb, d1, LpW, Lpb, CmW1, Cmb1, CmW2, Cmb2, CpW1, Cpb1, CpW2, Cpb2,
        LiW, Lib, CiW, Cib, iw1, ib1):
    return pl.pallas_call(
        _k3_body,
        grid=(N_PROBS,),
        in_specs=[
            pl.BlockSpec((CLAUSES_PER, 1), lambda p: (p, 0)),
            pl.BlockSpec((CLAUSES_PER, 1), lambda p: (p, 0)),
            pl.BlockSpec((CLAUSES_PER, 1), lambda p: (p, 0)),
            _full((R, DIM, DIM)), _full((R, DIM)),
            _full((R, DIM, DIM)), _full((R, DIM)),
            _full((R, DIM, DIM)), _full((R, DIM)),
            _full((R, DIM, DIM)), _full((R, DIM)),
            _full((R, DIM, DIM)), _full((R, DIM)),
            _full((1, DIM)), _full((1, DIM)), _full((1, DIM)), _full((1, DIM)),
            _full((1, DIM)), _full((1, DIM)),
        ],
        out_specs=pl.BlockSpec((CLAUSES_PER, DIM), lambda p: (p, 0)),
        out_shape=jax.ShapeDtypeStruct((N_CLAUSES, DIM), jnp.float32),
    )(ta, tb, d1, LpW, Lpb, CmW1, Cmb1, CmW2, Cmb2, CpW1, Cpb1, CpW2, Cpb2,
      LiW, Lib, CiW, Cib, iw1, ib1)


CHUNK_LITS = 25000
ACC_ROWS = CHUNK_LITS


@functools.partial(
    pl.kernel,
    out_type=jax.ShapeDtypeStruct((N_LITS, DIM), jnp.float32),
    mesh=_mesh,
    scratch_types=[
        pltpu.VMEM_SHARED((ACC_ROWS, DIM), jnp.float32),
        pltpu.VMEM((16, DIM), jnp.float32),
        pltpu.VMEM((5, 128), jnp.int32),
        pltpu.VMEM((5, 128), jnp.int32),
        pltpu.VMEM((5, 128), jnp.int32),
        pltpu.VMEM((5, 128), jnp.int32),
        pltpu.VMEM((128, DIM), jnp.float32),
        pltpu.VMEM((128, DIM), jnp.float32),
        pltpu.SemaphoreType.DMA,
        pltpu.SemaphoreType.DMA,
        pltpu.SemaphoreType.DMA,
        pltpu.SemaphoreType.DMA,
    ],
    compiler_params=pltpu.CompilerParams(use_tc_tiling_on_sc=False),
)
def _k4(cp_hbm, lit2d, cls2d, cl_out,
        acc_sh, zrows, litb, clsb, gidxb, offb, rows0, rows1,
        semg0, semg1, sems0, sems1):
    cid = lax.axis_index("c")
    sid = lax.axis_index("s")

    def zrow_body(i, _):
        z = jnp.zeros((16,), jnp.float32)
        for q in range(DIM // 16):
            zrows[i, pl.ds(q * 16, 16)] = z
        return 0

    lax.fori_loop(0, 16, zrow_body, 0)

    rows = [rows0, rows1]
    semg = [semg0, semg1]
    sems = [sems0, sems1]

    for kchunk in range(2):
        chunk = 2 * cid + kchunk
        lo = chunk * CHUNK_LITS

        def zblk(r, _):
            pltpu.sync_copy(zrows, acc_sh.at[pl.ds((sid + 16 * r) * 16, 16), :])
            return 0

        lax.fori_loop(0, 97, zblk, 0)
        @pl.when(sid < 10)
        def _():
            pltpu.sync_copy(zrows, acc_sh.at[pl.ds((1552 + sid) * 16, 16), :])
        @pl.when(sid == 15)
        def _():
            pltpu.sync_copy(zrows.at[pl.ds(0, 8), :],
                            acc_sh.at[pl.ds(24992, 8), :])
        plsc.subcore_barrier()

        def filter_row(i):
            for q in range(8):
                l16 = litb[i, pl.ds(q * 16, 16)]
                c16 = clsb[i, pl.ds(q * 16, 16)]
                inm = (l16 >= lo) & (l16 < lo + CHUNK_LITS)
                m1 = jnp.full((16,), -1, jnp.int32)
                gidxb[i, pl.ds(q * 16, 16)] = jnp.where(inm, c16, m1)
                offb[i, pl.ds(q * 16, 16)] = jnp.where(inm, l16 - lo, m1)

        def gath(i):
            return pltpu.async_copy(
                cp_hbm.at[plsc.Indices(gidxb.at[i], ignored_value=-1)],
                rows[i % 2], semg[i % 2])

        def scat(i):
            return pltpu.async_copy(
                rows[i % 2],
                acc_sh.at[plsc.Indices(offb.at[i], ignored_value=-1)],
                sems[i % 2], add=True)

        def do_batch(b, _):
            r0 = sid * 390 + 5 * b
            pltpu.sync_copy(lit2d.at[pl.ds(r0, 5), :], litb)
            pltpu.sync_copy(cls2d.at[pl.ds(r0, 5), :], clsb)
            for i in range(5):
                filter_row(i)
            g0 = gath(0)
            g1 = gath(1)
            g0.wait()
            s0 = scat(0)
            g1.wait()
            s1 = scat(1)
            s0.wait()
            g2 = gath(2)
            g2.wait()
            s2 = scat(2)
            s1.wait()
            g3 = gath(3)
            g3.wait()
            s3 = scat(3)
            s2.wait()
            g4 = gath(4)
            g4.wait()
            s4 = scat(4)
            s3.wait()
            s4.wait()
            return 0

        lax.fori_loop(0, 78, do_batch, 0)

        @pl.when(sid < EROWS % 16)
        def _():
            j = (EROWS // 16) * 16 + sid
            pltpu.sync_copy(lit2d.at[pl.ds(j, 1), :], litb.at[pl.ds(0, 1), :])
            pltpu.sync_copy(cls2d.at[pl.ds(j, 1), :], clsb.at[pl.ds(0, 1), :])
            filter_row(0)
            pltpu.sync_copy(
                cp_hbm.at[plsc.Indices(gidxb.at[0], ignored_value=-1)], rows0)
            pltpu.sync_copy(
                rows0, acc_sh.at[plsc.Indices(offb.at[0], ignored_value=-1)],
                add=True)
        plsc.subcore_barrier()

        def wb(b):
            r0 = b * 128
            pltpu.sync_copy(acc_sh.at[pl.ds(r0, 128), :], rows0)
            pltpu.sync_copy(rows0, cl_out.at[pl.ds(lo + r0, 128), :])

        def wb_round(r, _):
            wb(sid + 16 * r)
            return 0

        lax.fori_loop(0, 12, wb_round, 0)
        @pl.when(sid < 3)
        def _():
            wb(192 + sid)
        @pl.when(sid == 15)
        def _():
            pltpu.sync_copy(acc_sh.at[pl.ds(24960, 40), :],
                            rows0.at[pl.ds(0, 40), :])
            pltpu.sync_copy(rows0.at[pl.ds(0, 40), :],
                            cl_out.at[pl.ds(lo + 24960, 40), :])
        plsc.subcore_barrier()


def _k5_body(cl_ref, degf_ref, degs_ref, LiW_ref, Lib_ref,
             sums_ref, sumsq_ref, sumdeg_ref):
    p = pl.program_id(0)
    h = pl.program_id(1)
    L = cl_ref[...] + degf_ref[...] * LiW_ref[...] + Lib_ref[...]
    s1 = jnp.sum(L, axis=0, keepdims=True)
    s2 = jnp.sum(L * L, axis=0, keepdims=True)
    sd = jnp.sum(degs_ref[...]) * jnp.ones((1, DIM), jnp.float32)

    @pl.when(h == 0)
    def _():
        sums_ref[pl.ds(p, 1), :] = s1
        sumsq_ref[pl.ds(p, 1), :] = s2
        sumdeg_ref[pl.ds(p, 1), :] = sd

    @pl.when(h == 1)
    def _():
        sums_ref[pl.ds(p, 1), :] += s1
        sumsq_ref[pl.ds(p, 1), :] += s2
        sumdeg_ref[pl.ds(p, 1), :] += sd


def _k5(cl, deg1, LiW, Lib):
    return pl.pallas_call(
        _k5_body,
        grid=(N_PROBS, 2),
        in_specs=[
            pl.BlockSpec((VARS_PER, DIM), lambda p, h: (p + N_PROBS * h, 0)),
            pl.BlockSpec((VARS_PER, 1), lambda p, h: (p + N_PROBS * (1 - h), 0)),
            pl.BlockSpec((VARS_PER, 1), lambda p, h: (p + N_PROBS * h, 0)),
            _full((1, DIM)), _full((1, DIM)),
        ],
        out_specs=[
            _full((N_PROBS, DIM)),
            _full((N_PROBS, DIM)),
            _full((N_PROBS, DIM)),
        ],
        out_shape=[
            jax.ShapeDtypeStruct((N_PROBS, DIM), jnp.float32),
            jax.ShapeDtypeStruct((N_PROBS, DIM), jnp.float32),
            jax.ShapeDtypeStruct((N_PROBS, DIM), jnp.float32),
        ],
    )(cl, deg1, deg1, LiW, Lib)


def _k6_body(cl_ref, degf_ref, LiW_ref, Lib_ref, sums_ref, sumsq_ref,
             sumdeg_ref, iw2_ref, ib2_ref, LmW1_ref, Lmb1_ref, LmW2_ref,
             Lmb2_ref, LvW1_ref, Lvb1_ref, LvW2_ref, Lvb2_ref,
             out_ref, hacc_ref):
    p = pl.program_id(0)
    h = pl.program_id(1)
    ntot = float(2 * VARS_PER)
    mean = sums_ref[pl.ds(p, 1), :] / ntot
    var = sumsq_ref[pl.ds(p, 1), :] / ntot - mean * mean
    std = jnp.sqrt(var + EPS)
    w = LiW_ref[...]
    b = Lib_ref[...]
    L = cl_ref[...] + degf_ref[...] * w + b
    Ln = iw2_ref[...] * (L - mean) / std + ib2_ref[...]
    Hh = jnp.maximum(
        jnp.dot(Ln, LmW1_ref[R - 1], preferred_element_type=jnp.float32)
        + Lmb1_ref[R - 1][None], 0.0)
    hsum = jnp.sum(Hh, axis=0, keepdims=True)

    @pl.when(h == 0)
    def _():
        hacc_ref[...] = hsum

    @pl.when(h == 1)
    def _():
        Hbar = (hacc_ref[...] + hsum) / ntot
        rep = (jnp.dot(Hbar, LmW2_ref[R - 1], preferred_element_type=jnp.float32)
               + Lmb2_ref[R - 1][None]
               + (sumdeg_ref[pl.ds(p, 1), :] / ntot) * w + b)
        z = jnp.maximum(
            jnp.dot(rep, LvW1_ref[...], preferred_element_type=jnp.float32)
            + Lvb1_ref[...], 0.0)
        out_ref[pl.ds(p, 1), :] = (jnp.dot(z, LvW2_ref[...],
                                           preferred_element_type=jnp.float32)
                                   + Lvb2_ref[...])


def _k6(cl, deg1, LiW, Lib, sums, sumsq, sumdeg, iw2, ib2,
        LmW1, Lmb1, LmW2, Lmb2, LvW1, Lvb1, LvW2, Lvb2):
    return pl.pallas_call(
        _k6_body,
        grid=(N_PROBS, 2),
        in_specs=[
            pl.BlockSpec((VARS_PER, DIM), lambda p, h: (p + N_PROBS * h, 0)),
            pl.BlockSpec((VARS_PER, 1), lambda p, h: (p + N_PROBS * (1 - h), 0)),
            _full((1, DIM)), _full((1, DIM)),
            _full((N_PROBS, DIM)),
            _full((N_PROBS, DIM)),
            _full((N_PROBS, DIM)),
            _full((1, DIM)), _full((1, DIM)),
            _full((R, DIM, DIM)), _full((R, DIM)),
            _full((R, DIM, DIM)), _full((R, DIM)),
            _full((DIM, DIM)), _full((1, DIM)),
            _full((DIM, DIM)), _full((1, DIM)),
        ],
        out_specs=_full((N_PROBS, DIM)),
        out_shape=jax.ShapeDtypeStruct((N_PROBS, DIM), jnp.float32),
        scratch_shapes=[pltpu.VMEM((1, DIM), jnp.float32)],
    )(cl, deg1, LiW, Lib, sums, sumsq, sumdeg, iw2, ib2,
      LmW1, Lmb1, LmW2, Lmb2, LvW1, Lvb1, LvW2, Lvb2)


def kernel(lit_idx, clause_idx, L_init_W, L_init_b, C_init_W, C_init_b,
           Lp_W, Lp_b, Lm_W1, Lm_b1, Lm_W2, Lm_b2, Cm_W1, Cm_b1, Cm_W2,
           Cm_b2, Cp_W1, Cp_b1, Cp_W2, Cp_b2, in_w1, in_b1, in_w2, in_b2,
           Lv_W1, Lv_b1, Lv_W2, Lv_b2):
    lit2d = lit_idx.reshape(EROWS, 128)
    cls2d = clause_idx.reshape(EROWS, 128)

    deg_lit, deg_clause, t0, t1 = _k1(lit2d, cls2d)

    ta = t0.reshape(N_CLAUSES, 1)
    tb = t1.reshape(N_CLAUSES, 1)
    d1 = deg_clause.reshape(N_CLAUSES, 1)
    Lib = L_init_b.reshape(1, DIM)
    Cib = C_init_b.reshape(1, DIM)
    iw1 = in_w1.reshape(1, DIM)
    ib1 = in_b1.reshape(1, DIM)
    cp = _k3(ta, tb, d1, Lp_W, Lp_b, Cm_W1, Cm_b1, Cm_W2, Cm_b2,
             Cp_W1, Cp_b1, Cp_W2, Cp_b2, L_init_W, Lib, C_init_W, Cib,
             iw1, ib1)

    cl = _k4(cp, lit2d, cls2d)

    deg1 = deg_lit.reshape(N_LITS, 1)
    sums, sumsq, sumdeg = _k5(cl, deg1, L_init_W, Lib)
    out = _k6(cl, deg1, L_init_W, Lib, sums, sumsq, sumdeg,
              in_w2.reshape(1, DIM), in_b2.reshape(1, DIM),
              Lm_W1, Lm_b1, Lm_W2, Lm_b2,
              Lv_W1, Lv_b1.reshape(1, DIM), Lv_W2, Lv_b2.reshape(1, DIM))
    return out


# speedup vs baseline: 15.5380x; 1.3013x over previous
import functools

import jax
import jax.numpy as jnp
from jax import lax
from jax.experimental import pallas as pl
from jax.experimental.pallas import tpu as pltpu
from jax.experimental.pallas import tpu_sc as plsc

N_VARS = 50000
N_LITS = 100000
N_CLAUSES = 50000
N_CELLS = 800000
N_PROBS = 10
DIM = 64
R = 4
VARS_PER = N_VARS // N_PROBS
CLAUSES_PER = N_CLAUSES // N_PROBS

EROWS = N_CELLS // 128
EPS = 1e-6

_mesh = plsc.VectorSubcoreMesh(core_axis_name="c", subcore_axis_name="s")


def _zero_vmem_1d(ref, n):
    z = jnp.zeros((16,), jnp.float32)

    def body(i, _):
        ref[pl.ds(i * 16, 16)] = z
        return 0

    lax.fori_loop(0, n // 16, body, 0)


def _fill_vmem_1d(ref, n, value):
    v = jnp.full((16,), value, jnp.float32)

    def body(i, _):
        ref[pl.ds(i * 16, 16)] = v
        return 0

    lax.fori_loop(0, n // 16, body, 0)


def _spread_1d(sid, total, bs, fn):
    full = total // bs
    tail = total - full * bs
    rounds, rem = divmod(full, 16)
    for r in range(rounds):
        fn((sid + 16 * r) * bs, bs)
    if rem:
        @pl.when(sid < rem)
        def _():
            fn((16 * rounds + sid) * bs, bs)
    if tail:
        @pl.when(sid == 15)
        def _():
            fn(full * bs, tail)


def _edge_rows(sid, fn):
    def body(k, _):
        fn(sid + 16 * k)
        return 0

    lax.fori_loop(0, EROWS // 16, body, 0)
    rem = EROWS % 16
    if rem:
        @pl.when(sid < rem)
        def _():
            fn((EROWS // 16) * 16 + sid)


@functools.partial(
    pl.kernel,
    out_type=(
        jax.ShapeDtypeStruct((N_LITS,), jnp.float32),
        jax.ShapeDtypeStruct((N_CLAUSES,), jnp.float32),
        jax.ShapeDtypeStruct((N_CLAUSES,), jnp.float32),
        jax.ShapeDtypeStruct((N_CLAUSES,), jnp.float32),
    ),
    mesh=_mesh,
    scratch_types=[
        pltpu.VMEM_SHARED((N_LITS,), jnp.float32),
        pltpu.VMEM_SHARED((N_CLAUSES,), jnp.float32),
        pltpu.VMEM_SHARED((N_CLAUSES,), jnp.float32),
        pltpu.VMEM((2048,), jnp.float32),
        pltpu.VMEM((2000,), jnp.float32),
        pltpu.VMEM((128,), jnp.float32),
        pltpu.VMEM((5, 128), jnp.int32),
        pltpu.VMEM((5, 128), jnp.int32),
        pltpu.VMEM((5, 128), jnp.float32),
        pltpu.SemaphoreType.DMA,
        pltpu.SemaphoreType.DMA,
    ],
    compiler_params=pltpu.CompilerParams(use_tc_tiling_on_sc=False),
)
def _k1(lit2d, cls2d, deg_lit_out, deg_cl_out, t0_out, t1_out,
        deg_lit_sh, deg_cl_sh, t_sh, zbuf, stage, ones_v, litb, clsb, valb,
        semA, semB):
    cid = lax.axis_index("c")
    sid = lax.axis_index("s")

    _zero_vmem_1d(zbuf, 2048)
    _fill_vmem_1d(ones_v, 128, 1.0)

    def zero_to(sh):
        def fn(off, size):
            pltpu.sync_copy(zbuf.at[pl.ds(0, size)], sh.at[pl.ds(off, size)])
        return fn

    _spread_1d(sid, N_LITS, 2000, zero_to(deg_lit_sh))
    _spread_1d(sid, N_CLAUSES, 2000, zero_to(deg_cl_sh))
    _spread_1d(sid, N_CLAUSES, 2000, zero_to(t_sh))
    plsc.subcore_barrier()

    def hist_batch(b, _):
        r0 = sid * 390 + 5 * b
        pltpu.sync_copy(lit2d.at[pl.ds(r0, 5), :], litb)
        pltpu.sync_copy(cls2d.at[pl.ds(r0, 5), :], clsb)
        ds = []
        for i in range(5):
            ds.append(pltpu.async_copy(
                ones_v, deg_lit_sh.at[litb.at[i]], semA, add=True))
            ds.append(pltpu.async_copy(
                ones_v, deg_cl_sh.at[clsb.at[i]], semB, add=True))
        for d in ds:
            d.wait()
        return 0

    lax.fori_loop(0, 78, hist_batch, 0)
    @pl.when(sid < EROWS % 16)
    def _():
        j = (EROWS // 16) * 16 + sid
        pltpu.sync_copy(lit2d.at[pl.ds(j, 1), :], litb.at[pl.ds(0, 1), :])
        pltpu.sync_copy(cls2d.at[pl.ds(j, 1), :], clsb.at[pl.ds(0, 1), :])
        pltpu.sync_copy(ones_v, deg_lit_sh.at[litb.at[0]], add=True)
        pltpu.sync_copy(ones_v, deg_cl_sh.at[clsb.at[0]], add=True)
    plsc.subcore_barrier()

    base = cid * (EROWS // 2) + sid * 195

    def t_batch(b, _):
        r0 = base + 5 * b
        pltpu.sync_copy(lit2d.at[pl.ds(r0, 5), :], litb)
        pltpu.sync_copy(cls2d.at[pl.ds(r0, 5), :], clsb)
        gs = [pltpu.async_copy(deg_lit_sh.at[litb.at[i]], valb.at[i], semA)
              for i in range(5)]
        for g in gs:
            g.wait()
        ss = [pltpu.async_copy(valb.at[i], t_sh.at[clsb.at[i]], semB, add=True)
              for i in range(5)]
        for s in ss:
            s.wait()
        return 0

    lax.fori_loop(0, 39, t_batch, 0)
    @pl.when(sid < 5)
    def _():
        j = cid * (EROWS // 2) + 3120 + sid
        pltpu.sync_copy(lit2d.at[pl.ds(j, 1), :], litb.at[pl.ds(0, 1), :])
        pltpu.sync_copy(cls2d.at[pl.ds(j, 1), :], clsb.at[pl.ds(0, 1), :])
        pltpu.sync_copy(deg_lit_sh.at[litb.at[0]], valb.at[0])
        pltpu.sync_copy(valb.at[0], t_sh.at[clsb.at[0]], add=True)
    plsc.subcore_barrier()

    def publish(sh, out):
        def fn(off, size):
            pltpu.sync_copy(sh.at[pl.ds(off, size)], stage.at[pl.ds(0, size)])
            pltpu.sync_copy(stage.at[pl.ds(0, size)], out.at[pl.ds(off, size)])
        return fn

    @pl.when(cid == 0)
    def _():
        _spread_1d(sid, N_LITS, 2000, publish(deg_lit_sh, deg_lit_out))
        _spread_1d(sid, N_CLAUSES, 2000, publish(deg_cl_sh, deg_cl_out))
        _spread_1d(sid, N_CLAUSES, 2000, publish(t_sh, t0_out))

    @pl.when(cid == 1)
    def _():
        _spread_1d(sid, N_CLAUSES, 2000, publish(t_sh, t1_out))


def _k3_body(ta_ref, tb_ref, d1_ref, LpW_ref, Lpb_ref, CmW1_ref, Cmb1_ref,
             CmW2_ref, Cmb2_ref, CpW1_ref, Cpb1_ref, CpW2_ref, Cpb2_ref,
             LiW_ref, Lib_ref, CiW_ref, Cib_ref, iw1_ref, ib1_ref, out_ref):
    n = float(CLAUSES_PER)
    t_col = ta_ref[...] + tb_ref[...]
    d_col = d1_ref[...]
    St = jnp.sum(t_col) / n
    Sd = jnp.sum(d_col) / n
    Vt = jnp.sum(t_col * t_col) / n - St * St
    Vd = jnp.sum(d_col * d_col) / n - Sd * Sd
    Ctd = jnp.sum(t_col * d_col) / n - St * Sd
    w = LiW_ref[...]
    b = Lib_ref[...]
    iw1 = iw1_ref[...]
    ib1 = ib1_ref[...]
    acc = d_col * CiW_ref[...] + Cib_ref[...]
    for i in range(R):
        Wp = LpW_ref[i]
        u = jnp.dot(w, Wp, preferred_element_type=jnp.float32)
        v = jnp.dot(b, Wp, preferred_element_type=jnp.float32) + Lpb_ref[i][None]
        var = Vt * u * u + 2.0 * Ctd * u * v + Vd * v * v
        std = jnp.sqrt(var + EPS)
        ai = iw1 * u / std
        bi = iw1 * v / std
        ei = ib1 - (St * u + Sd * v) * iw1 / std
        A = jnp.dot(ai, CmW1_ref[i], preferred_element_type=jnp.float32)
        B = jnp.dot(bi, CmW1_ref[i], preferred_element_type=jnp.float32)
        E = (jnp.dot(ei, CmW1_ref[i], preferred_element_type=jnp.float32)
             + Cmb1_ref[i][None])
        H = jnp.maximum(t_col * A + d_col * B + E, 0.0)
        acc = (acc + jnp.dot(H, CmW2_ref[i], preferred_element_type=jnp.float32)
               + Cmb2_ref[i][None])
    z = jnp.maximum(
        jnp.dot(acc, CpW1_ref[R - 1], preferred_element_type=jnp.float32)
        + Cpb1_ref[R - 1][None], 0.0)
    out_ref[...] = (jnp.dot(z, CpW2_ref[R - 1], preferred_element_type=jnp.float32)
                    + Cpb2_ref[R - 1][None])


def _full(shape):
    return pl.BlockSpec(shape, lambda *args: tuple(0 for _ in shape))


def _k3(ta, tb, d1, LpW, Lpb, CmW1, Cmb1, CmW2, Cmb2, CpW1, Cpb1, CpW2, Cpb2,
        LiW, Lib, CiW, Cib, iw1, ib1):
    return pl.pallas_call(
        _k3_body,
        grid=(N_PROBS,),
        in_specs=[
            pl.BlockSpec((CLAUSES_PER, 1), lambda p: (p, 0)),
            pl.BlockSpec((CLAUSES_PER, 1), lambda p: (p, 0)),
            pl.BlockSpec((CLAUSES_PER, 1), lambda p: (p, 0)),
            _full((R, DIM, DIM)), _full((R, DIM)),
            _full((R, DIM, DIM)), _full((R, DIM)),
            _full((R, DIM, DIM)), _full((R, DIM)),
            _full((R, DIM, DIM)), _full((R, DIM)),
            _full((R, DIM, DIM)), _full((R, DIM)),
            _full((1, DIM)), _full((1, DIM)), _full((1, DIM)), _full((1, DIM)),
            _full((1, DIM)), _full((1, DIM)),
        ],
        out_specs=pl.BlockSpec((CLAUSES_PER, DIM), lambda p: (p, 0)),
        out_shape=jax.ShapeDtypeStruct((N_CLAUSES, DIM), jnp.float32),
    )(ta, tb, d1, LpW, Lpb, CmW1, Cmb1, CmW2, Cmb2, CpW1, Cpb1, CpW2, Cpb2,
      LiW, Lib, CiW, Cib, iw1, ib1)


CHUNK_LITS = 25000
ACC_ROWS = CHUNK_LITS


@functools.partial(
    pl.kernel,
    out_type=jax.ShapeDtypeStruct((N_LITS, DIM), jnp.float32),
    mesh=_mesh,
    scratch_types=[
        pltpu.VMEM_SHARED((ACC_ROWS, DIM), jnp.float32),
        pltpu.VMEM((16, DIM), jnp.float32),
        pltpu.VMEM((10, 128), jnp.int32),
        pltpu.VMEM((10, 128), jnp.int32),
        pltpu.VMEM((10, 128), jnp.int32),
        pltpu.VMEM((10, 128), jnp.int32),
        pltpu.VMEM((128, DIM), jnp.float32),
        pltpu.VMEM((128, DIM), jnp.float32),
        pltpu.VMEM((128, DIM), jnp.float32),
        pltpu.SemaphoreType.DMA,
        pltpu.SemaphoreType.DMA,
        pltpu.SemaphoreType.DMA,
        pltpu.SemaphoreType.DMA,
        pltpu.SemaphoreType.DMA,
        pltpu.SemaphoreType.DMA,
    ],
    compiler_params=pltpu.CompilerParams(use_tc_tiling_on_sc=False),
)
def _k4(cp_hbm, lit2d, cls2d, cl_out,
        acc_sh, zrows, litb, clsb, gidxb, offb, rows0, rows1, rows2,
        semg0, semg1, semg2, sems0, sems1, sems2):
    cid = lax.axis_index("c")
    sid = lax.axis_index("s")

    def zrow_body(i, _):
        z = jnp.zeros((16,), jnp.float32)
        for q in range(DIM // 16):
            zrows[i, pl.ds(q * 16, 16)] = z
        return 0

    lax.fori_loop(0, 16, zrow_body, 0)

    rows = [rows0, rows1, rows2]
    semg = [semg0, semg1, semg2]
    sems = [sems0, sems1, sems2]

    for kchunk in range(2):
        chunk = 2 * cid + kchunk
        lo = chunk * CHUNK_LITS

        def zblk(r, _):
            pltpu.sync_copy(zrows, acc_sh.at[pl.ds((sid + 16 * r) * 16, 16), :])
            return 0

        lax.fori_loop(0, 97, zblk, 0)
        @pl.when(sid < 10)
        def _():
            pltpu.sync_copy(zrows, acc_sh.at[pl.ds((1552 + sid) * 16, 16), :])
        @pl.when(sid == 15)
        def _():
            pltpu.sync_copy(zrows.at[pl.ds(0, 8), :],
                            acc_sh.at[pl.ds(24992, 8), :])
        plsc.subcore_barrier()

        def filter_row(i):
            for q in range(8):
                l16 = litb[i, pl.ds(q * 16, 16)]
                c16 = clsb[i, pl.ds(q * 16, 16)]
                inm = (l16 >= lo) & (l16 < lo + CHUNK_LITS)
                m1 = jnp.full((16,), -1, jnp.int32)
                gidxb[i, pl.ds(q * 16, 16)] = jnp.where(inm, c16, m1)
                offb[i, pl.ds(q * 16, 16)] = jnp.where(inm, l16 - lo, m1)

        def gath(i):
            return pltpu.async_copy(
                cp_hbm.at[plsc.Indices(gidxb.at[i], ignored_value=-1)],
                rows[i % 3], semg[i % 3])

        def scat(i):
            return pltpu.async_copy(
                rows[i % 3],
                acc_sh.at[plsc.Indices(offb.at[i], ignored_value=-1)],
                sems[i % 3], add=True)

        B = 10

        def do_batch(b, _):
            r0 = sid * 390 + B * b
            pltpu.sync_copy(lit2d.at[pl.ds(r0, B), :], litb)
            pltpu.sync_copy(cls2d.at[pl.ds(r0, B), :], clsb)
            for i in range(B):
                filter_row(i)
            gs = {0: gath(0), 1: gath(1)}
            ss = {}
            for i in range(B):
                gs[i % 3].wait()
                ss[i % 3] = scat(i)
                if i + 2 < B:
                    if i >= 1:
                        ss[(i + 2) % 3].wait()
                    gs[(i + 2) % 3] = gath(i + 2)
            for sl in range(3):
                ss[sl].wait()
            return 0

        lax.fori_loop(0, 390 // B, do_batch, 0)

        @pl.when(sid < EROWS % 16)
        def _():
            j = (EROWS // 16) * 16 + sid
            pltpu.sync_copy(lit2d.at[pl.ds(j, 1), :], litb.at[pl.ds(0, 1), :])
            pltpu.sync_copy(cls2d.at[pl.ds(j, 1), :], clsb.at[pl.ds(0, 1), :])
            filter_row(0)
            pltpu.sync_copy(
                cp_hbm.at[plsc.Indices(gidxb.at[0], ignored_value=-1)], rows0)
            pltpu.sync_copy(
                rows0, acc_sh.at[plsc.Indices(offb.at[0], ignored_value=-1)],
                add=True)
        plsc.subcore_barrier()

        def wb(b):
            r0 = b * 128
            pltpu.sync_copy(acc_sh.at[pl.ds(r0, 128), :], rows0)
            pltpu.sync_copy(rows0, cl_out.at[pl.ds(lo + r0, 128), :])

        def wb_round(r, _):
            wb(sid + 16 * r)
            return 0

        lax.fori_loop(0, 12, wb_round, 0)
        @pl.when(sid < 3)
        def _():
            wb(192 + sid)
        @pl.when(sid == 15)
        def _():
            pltpu.sync_copy(acc_sh.at[pl.ds(24960, 40), :],
                            rows0.at[pl.ds(0, 40), :])
            pltpu.sync_copy(rows0.at[pl.ds(0, 40), :],
                            cl_out.at[pl.ds(lo + 24960, 40), :])
        plsc.subcore_barrier()


def _k56_body(cl_ref, degf_ref, degs_ref, LiW_ref, Lib_ref, iw2_ref, ib2_ref,
              LmW1_ref, Lmb1_ref, LmW2_ref, Lmb2_ref, LvW1_ref, Lvb1_ref,
              LvW2_ref, Lvb2_ref, out_ref, sums_ref, sumsq_ref, sumdeg_ref,
              hacc_ref):
    ph = pl.program_id(0)
    p = pl.program_id(1)
    h = pl.program_id(2)
    ntot = float(2 * VARS_PER)
    w = LiW_ref[...]
    b = Lib_ref[...]
    L = cl_ref[...] + degf_ref[...] * w + b

    @pl.when(ph == 0)
    def _():
        s1 = jnp.sum(L, axis=0, keepdims=True)
        s2 = jnp.sum(L * L, axis=0, keepdims=True)
        sd = jnp.sum(degs_ref[...]) * jnp.ones((1, DIM), jnp.float32)

        @pl.when(h == 0)
        def _():
            sums_ref[pl.ds(p, 1), :] = s1
            sumsq_ref[pl.ds(p, 1), :] = s2
            sumdeg_ref[pl.ds(p, 1), :] = sd

        @pl.when(h == 1)
        def _():
            sums_ref[pl.ds(p, 1), :] += s1
            sumsq_ref[pl.ds(p, 1), :] += s2
            sumdeg_ref[pl.ds(p, 1), :] += sd

    @pl.when(ph == 1)
    def _():
        mean = sums_ref[pl.ds(p, 1), :] / ntot
        var = sumsq_ref[pl.ds(p, 1), :] / ntot - mean * mean
        std = jnp.sqrt(var + EPS)
        Ln = iw2_ref[...] * (L - mean) / std + ib2_ref[...]
        Hh = jnp.maximum(
            jnp.dot(Ln, LmW1_ref[R - 1], preferred_element_type=jnp.float32)
            + Lmb1_ref[R - 1][None], 0.0)
        hsum = jnp.sum(Hh, axis=0, keepdims=True)

        @pl.when(h == 0)
        def _():
            hacc_ref[...] = hsum

        @pl.when(h == 1)
        def _():
            Hbar = (hacc_ref[...] + hsum) / ntot
            rep = (jnp.dot(Hbar, LmW2_ref[R - 1],
                           preferred_element_type=jnp.float32)
                   + Lmb2_ref[R - 1][None]
                   + (sumdeg_ref[pl.ds(p, 1), :] / ntot) * w + b)
            z = jnp.maximum(
                jnp.dot(rep, LvW1_ref[...], preferred_element_type=jnp.float32)
                + Lvb1_ref[...], 0.0)
            out_ref[pl.ds(p, 1), :] = (
                jnp.dot(z, LvW2_ref[...], preferred_element_type=jnp.float32)
                + Lvb2_ref[...])


def _k56(cl, deg1, LiW, Lib, iw2, ib2,
         LmW1, Lmb1, LmW2, Lmb2, LvW1, Lvb1, LvW2, Lvb2):
    return pl.pallas_call(
        _k56_body,
        grid=(2, N_PROBS, 2),
        in_specs=[
            pl.BlockSpec((VARS_PER, DIM),
                         lambda ph, p, h: (p + N_PROBS * h, 0)),
            pl.BlockSpec((VARS_PER, 1),
                         lambda ph, p, h: (p + N_PROBS * (1 - h), 0)),
            pl.BlockSpec((VARS_PER, 1),
                         lambda ph, p, h: (p + N_PROBS * h, 0)),
            _full((1, DIM)), _full((1, DIM)),
            _full((1, DIM)), _full((1, DIM)),
            _full((R, DIM, DIM)), _full((R, DIM)),
            _full((R, DIM, DIM)), _full((R, DIM)),
            _full((DIM, DIM)), _full((1, DIM)),
            _full((DIM, DIM)), _full((1, DIM)),
        ],
        out_specs=_full((N_PROBS, DIM)),
        out_shape=jax.ShapeDtypeStruct((N_PROBS, DIM), jnp.float32),
        scratch_shapes=[
            pltpu.VMEM((N_PROBS, DIM), jnp.float32),
            pltpu.VMEM((N_PROBS, DIM), jnp.float32),
            pltpu.VMEM((N_PROBS, DIM), jnp.float32),
            pltpu.VMEM((1, DIM), jnp.float32),
        ],
    )(cl, deg1, deg1, LiW, Lib, iw2, ib2,
      LmW1, Lmb1, LmW2, Lmb2, LvW1, Lvb1, LvW2, Lvb2)


def kernel(lit_idx, clause_idx, L_init_W, L_init_b, C_init_W, C_init_b,
           Lp_W, Lp_b, Lm_W1, Lm_b1, Lm_W2, Lm_b2, Cm_W1, Cm_b1, Cm_W2,
           Cm_b2, Cp_W1, Cp_b1, Cp_W2, Cp_b2, in_w1, in_b1, in_w2, in_b2,
           Lv_W1, Lv_b1, Lv_W2, Lv_b2):
    lit2d = lit_idx.reshape(EROWS, 128)
    cls2d = clause_idx.reshape(EROWS, 128)

    deg_lit, deg_clause, t0, t1 = _k1(lit2d, cls2d)

    ta = t0.reshape(N_CLAUSES, 1)
    tb = t1.reshape(N_CLAUSES, 1)
    d1 = deg_clause.reshape(N_CLAUSES, 1)
    Lib = L_init_b.reshape(1, DIM)
    Cib = C_init_b.reshape(1, DIM)
    iw1 = in_w1.reshape(1, DIM)
    ib1 = in_b1.reshape(1, DIM)
    cp = _k3(ta, tb, d1, Lp_W, Lp_b, Cm_W1, Cm_b1, Cm_W2, Cm_b2,
             Cp_W1, Cp_b1, Cp_W2, Cp_b2, L_init_W, Lib, C_init_W, Cib,
             iw1, ib1)

    cl = _k4(cp, lit2d, cls2d)

    deg1 = deg_lit.reshape(N_LITS, 1)
    out = _k56(cl, deg1, L_init_W, Lib,
               in_w2.reshape(1, DIM), in_b2.reshape(1, DIM),
               Lm_W1, Lm_b1, Lm_W2, Lm_b2,
               Lv_W1, Lv_b1.reshape(1, DIM), Lv_W2, Lv_b2.reshape(1, DIM))
    return out


# speedup vs baseline: 16.7961x; 1.0810x over previous
import functools

import jax
import jax.numpy as jnp
from jax import lax
from jax.experimental import pallas as pl
from jax.experimental.pallas import tpu as pltpu
from jax.experimental.pallas import tpu_sc as plsc

N_VARS = 50000
N_LITS = 100000
N_CLAUSES = 50000
N_CELLS = 800000
N_PROBS = 10
DIM = 64
R = 4
VARS_PER = N_VARS // N_PROBS
CLAUSES_PER = N_CLAUSES // N_PROBS

EROWS = N_CELLS // 128
EPS = 1e-6

_mesh = plsc.VectorSubcoreMesh(core_axis_name="c", subcore_axis_name="s")


def _zero_vmem_1d(ref, n):
    z = jnp.zeros((16,), jnp.float32)

    def body(i, _):
        ref[pl.ds(i * 16, 16)] = z
        return 0

    lax.fori_loop(0, n // 16, body, 0)


def _fill_vmem_1d(ref, n, value):
    v = jnp.full((16,), value, jnp.float32)

    def body(i, _):
        ref[pl.ds(i * 16, 16)] = v
        return 0

    lax.fori_loop(0, n // 16, body, 0)


def _spread_1d(sid, total, bs, fn):
    full = total // bs
    tail = total - full * bs
    rounds, rem = divmod(full, 16)
    for r in range(rounds):
        fn((sid + 16 * r) * bs, bs)
    if rem:
        @pl.when(sid < rem)
        def _():
            fn((16 * rounds + sid) * bs, bs)
    if tail:
        @pl.when(sid == 15)
        def _():
            fn(full * bs, tail)


def _edge_rows(sid, fn):
    def body(k, _):
        fn(sid + 16 * k)
        return 0

    lax.fori_loop(0, EROWS // 16, body, 0)
    rem = EROWS % 16
    if rem:
        @pl.when(sid < rem)
        def _():
            fn((EROWS // 16) * 16 + sid)


@functools.partial(
    pl.kernel,
    out_type=(
        jax.ShapeDtypeStruct((N_LITS,), jnp.float32),
        jax.ShapeDtypeStruct((N_CLAUSES,), jnp.float32),
        jax.ShapeDtypeStruct((N_CLAUSES,), jnp.float32),
        jax.ShapeDtypeStruct((N_CLAUSES,), jnp.float32),
    ),
    mesh=_mesh,
    scratch_types=[
        pltpu.VMEM_SHARED((N_LITS,), jnp.float32),
        pltpu.VMEM_SHARED((N_CLAUSES,), jnp.float32),
        pltpu.VMEM_SHARED((N_CLAUSES,), jnp.float32),
        pltpu.VMEM((2048,), jnp.float32),
        pltpu.VMEM((2000,), jnp.float32),
        pltpu.VMEM((128,), jnp.float32),
        pltpu.VMEM((13, 128), jnp.int32),
        pltpu.VMEM((13, 128), jnp.int32),
        pltpu.VMEM((13, 128), jnp.float32),
        pltpu.SemaphoreType.DMA,
        pltpu.SemaphoreType.DMA,
    ],
    compiler_params=pltpu.CompilerParams(use_tc_tiling_on_sc=False),
)
def _k1(lit2d, cls2d, deg_lit_out, deg_cl_out, t0_out, t1_out,
        deg_lit_sh, deg_cl_sh, t_sh, zbuf, stage, ones_v, litb, clsb, valb,
        semA, semB):
    cid = lax.axis_index("c")
    sid = lax.axis_index("s")

    _zero_vmem_1d(zbuf, 2048)
    _fill_vmem_1d(ones_v, 128, 1.0)

    def zero_to(sh):
        def fn(off, size):
            pltpu.sync_copy(zbuf.at[pl.ds(0, size)], sh.at[pl.ds(off, size)])
        return fn

    _spread_1d(sid, N_LITS, 2000, zero_to(deg_lit_sh))
    _spread_1d(sid, N_CLAUSES, 2000, zero_to(deg_cl_sh))
    _spread_1d(sid, N_CLAUSES, 2000, zero_to(t_sh))
    plsc.subcore_barrier()

    def hist_batch(b, _):
        r0 = sid * 390 + 13 * b
        pltpu.sync_copy(lit2d.at[pl.ds(r0, 13), :], litb)
        pltpu.sync_copy(cls2d.at[pl.ds(r0, 13), :], clsb)
        ds = []
        for i in range(13):
            ds.append(pltpu.async_copy(
                ones_v, deg_lit_sh.at[litb.at[i]], semA, add=True))
            ds.append(pltpu.async_copy(
                ones_v, deg_cl_sh.at[clsb.at[i]], semB, add=True))
        for d in ds:
            d.wait()
        return 0

    lax.fori_loop(0, 30, hist_batch, 0)
    @pl.when(sid < EROWS % 16)
    def _():
        j = (EROWS // 16) * 16 + sid
        pltpu.sync_copy(lit2d.at[pl.ds(j, 1), :], litb.at[pl.ds(0, 1), :])
        pltpu.sync_copy(cls2d.at[pl.ds(j, 1), :], clsb.at[pl.ds(0, 1), :])
        pltpu.sync_copy(ones_v, deg_lit_sh.at[litb.at[0]], add=True)
        pltpu.sync_copy(ones_v, deg_cl_sh.at[clsb.at[0]], add=True)
    plsc.subcore_barrier()

    base = cid * (EROWS // 2) + sid * 195

    def t_batch(b, _):
        r0 = base + 13 * b
        pltpu.sync_copy(lit2d.at[pl.ds(r0, 13), :], litb)
        pltpu.sync_copy(cls2d.at[pl.ds(r0, 13), :], clsb)
        gs = [pltpu.async_copy(deg_lit_sh.at[litb.at[i]], valb.at[i], semA)
              for i in range(13)]
        for g in gs:
            g.wait()
        ss = [pltpu.async_copy(valb.at[i], t_sh.at[clsb.at[i]], semB, add=True)
              for i in range(13)]
        for s in ss:
            s.wait()
        return 0

    lax.fori_loop(0, 15, t_batch, 0)
    @pl.when(sid < 5)
    def _():
        j = cid * (EROWS // 2) + 3120 + sid
        pltpu.sync_copy(lit2d.at[pl.ds(j, 1), :], litb.at[pl.ds(0, 1), :])
        pltpu.sync_copy(cls2d.at[pl.ds(j, 1), :], clsb.at[pl.ds(0, 1), :])
        pltpu.sync_copy(deg_lit_sh.at[litb.at[0]], valb.at[0])
        pltpu.sync_copy(valb.at[0], t_sh.at[clsb.at[0]], add=True)
    plsc.subcore_barrier()

    def publish(sh, out):
        def fn(off, size):
            pltpu.sync_copy(sh.at[pl.ds(off, size)], stage.at[pl.ds(0, size)])
            pltpu.sync_copy(stage.at[pl.ds(0, size)], out.at[pl.ds(off, size)])
        return fn

    @pl.when(cid == 0)
    def _():
        _spread_1d(sid, N_LITS, 2000, publish(deg_lit_sh, deg_lit_out))
        _spread_1d(sid, N_CLAUSES, 2000, publish(deg_cl_sh, deg_cl_out))
        _spread_1d(sid, N_CLAUSES, 2000, publish(t_sh, t0_out))

    @pl.when(cid == 1)
    def _():
        _spread_1d(sid, N_CLAUSES, 2000, publish(t_sh, t1_out))


def _k3_body(ta_ref, tb_ref, d1_ref, LpW_ref, Lpb_ref, CmW1_ref, Cmb1_ref,
             CmW2_ref, Cmb2_ref, CpW1_ref, Cpb1_ref, CpW2_ref, Cpb2_ref,
             LiW_ref, Lib_ref, CiW_ref, Cib_ref, iw1_ref, ib1_ref, out_ref):
    n = float(CLAUSES_PER)
    t_col = ta_ref[...] + tb_ref[...]
    d_col = d1_ref[...]
    St = jnp.sum(t_col) / n
    Sd = jnp.sum(d_col) / n
    Vt = jnp.sum(t_col * t_col) / n - St * St
    Vd = jnp.sum(d_col * d_col) / n - Sd * Sd
    Ctd = jnp.sum(t_col * d_col) / n - St * Sd
    w = LiW_ref[...]
    b = Lib_ref[...]
    iw1 = iw1_ref[...]
    ib1 = ib1_ref[...]
    acc = d_col * CiW_ref[...] + Cib_ref[...]
    for i in range(R):
        Wp = LpW_ref[i]
        u = jnp.dot(w, Wp, preferred_element_type=jnp.float32)
        v = jnp.dot(b, Wp, preferred_element_type=jnp.float32) + Lpb_ref[i][None]
        var = Vt * u * u + 2.0 * Ctd * u * v + Vd * v * v
        std = jnp.sqrt(var + EPS)
        ai = iw1 * u / std
        bi = iw1 * v / std
        ei = ib1 - (St * u + Sd * v) * iw1 / std
        A = jnp.dot(ai, CmW1_ref[i], preferred_element_type=jnp.float32)
        B = jnp.dot(bi, CmW1_ref[i], preferred_element_type=jnp.float32)
        E = (jnp.dot(ei, CmW1_ref[i], preferred_element_type=jnp.float32)
             + Cmb1_ref[i][None])
        H = jnp.maximum(t_col * A + d_col * B + E, 0.0)
        acc = (acc + jnp.dot(H, CmW2_ref[i], preferred_element_type=jnp.float32)
               + Cmb2_ref[i][None])
    z = jnp.maximum(
        jnp.dot(acc, CpW1_ref[R - 1], preferred_element_type=jnp.float32)
        + Cpb1_ref[R - 1][None], 0.0)
    out_ref[...] = (jnp.dot(z, CpW2_ref[R - 1], preferred_element_type=jnp.float32)
                    + Cpb2_ref[R - 1][None])


def _full(shape):
    return pl.BlockSpec(shape, lambda *args: tuple(0 for _ in shape))


def _k3(ta, tb, d1, LpW, Lpb, CmW1, Cmb1, CmW2, Cmb2, CpW1, Cpb1, CpW2, Cpb2,
        LiW, Lib, CiW, Cib, iw1, ib1):
    return pl.pallas_call(
        _k3_body,
        grid=(N_PROBS,),
        in_specs=[
            pl.BlockSpec((CLAUSES_PER, 1), lambda p: (p, 0)),
            pl.BlockSpec((CLAUSES_PER, 1), lambda p: (p, 0)),
            pl.BlockSpec((CLAUSES_PER, 1), lambda p: (p, 0)),
            _full((R, DIM, DIM)), _full((R, DIM)),
            _full((R, DIM, DIM)), _full((R, DIM)),
            _full((R, DIM, DIM)), _full((R, DIM)),
            _full((R, DIM, DIM)), _full((R, DIM)),
            _full((R, DIM, DIM)), _full((R, DIM)),
            _full((1, DIM)), _full((1, DIM)), _full((1, DIM)), _full((1, DIM)),
            _full((1, DIM)), _full((1, DIM)),
        ],
        out_specs=pl.BlockSpec((CLAUSES_PER, DIM), lambda p: (p, 0)),
        out_shape=jax.ShapeDtypeStruct((N_CLAUSES, DIM), jnp.float32),
    )(ta, tb, d1, LpW, Lpb, CmW1, Cmb1, CmW2, Cmb2, CpW1, Cpb1, CpW2, Cpb2,
      LiW, Lib, CiW, Cib, iw1, ib1)


CHUNK_LITS = 25000
ACC_ROWS = CHUNK_LITS


@functools.partial(
    pl.kernel,
    out_type=jax.ShapeDtypeStruct((N_LITS, DIM), jnp.float32),
    mesh=_mesh,
    scratch_types=[
        pltpu.VMEM_SHARED((ACC_ROWS, DIM), jnp.float32),
        pltpu.VMEM((16, DIM), jnp.float32),
        pltpu.VMEM((10, 128), jnp.int32),
        pltpu.VMEM((10, 128), jnp.int32),
        pltpu.VMEM((10, 128), jnp.int32),
        pltpu.VMEM((10, 128), jnp.int32),
        pltpu.VMEM((128, DIM), jnp.float32),
        pltpu.VMEM((128, DIM), jnp.float32),
        pltpu.VMEM((128, DIM), jnp.float32),
        pltpu.SemaphoreType.DMA,
        pltpu.SemaphoreType.DMA,
        pltpu.SemaphoreType.DMA,
        pltpu.SemaphoreType.DMA,
        pltpu.SemaphoreType.DMA,
        pltpu.SemaphoreType.DMA,
    ],
    compiler_params=pltpu.CompilerParams(use_tc_tiling_on_sc=False),
)
def _k4(cp_hbm, lit2d, cls2d, cl_out,
        acc_sh, zrows, litb, clsb, gidxb, offb, rows0, rows1, rows2,
        semg0, semg1, semg2, sems0, sems1, sems2):
    cid = lax.axis_index("c")
    sid = lax.axis_index("s")

    def zrow_body(i, _):
        z = jnp.zeros((16,), jnp.float32)
        for q in range(DIM // 16):
            zrows[i, pl.ds(q * 16, 16)] = z
        return 0

    lax.fori_loop(0, 16, zrow_body, 0)

    rows = [rows0, rows1, rows2]
    semg = [semg0, semg1, semg2]
    sems = [sems0, sems1, sems2]

    for kchunk in range(2):
        chunk = 2 * cid + kchunk
        lo = chunk * CHUNK_LITS

        def zblk(r, _):
            pltpu.sync_copy(zrows, acc_sh.at[pl.ds((sid + 16 * r) * 16, 16), :])
            return 0

        lax.fori_loop(0, 97, zblk, 0)
        @pl.when(sid < 10)
        def _():
            pltpu.sync_copy(zrows, acc_sh.at[pl.ds((1552 + sid) * 16, 16), :])
        @pl.when(sid == 15)
        def _():
            pltpu.sync_copy(zrows.at[pl.ds(0, 8), :],
                            acc_sh.at[pl.ds(24992, 8), :])
        plsc.subcore_barrier()

        def filter_row(i):
            for q in range(8):
                l16 = litb[i, pl.ds(q * 16, 16)]
                c16 = clsb[i, pl.ds(q * 16, 16)]
                inm = (l16 >= lo) & (l16 < lo + CHUNK_LITS)
                m1 = jnp.full((16,), -1, jnp.int32)
                gidxb[i, pl.ds(q * 16, 16)] = jnp.where(inm, c16, m1)
                offb[i, pl.ds(q * 16, 16)] = jnp.where(inm, l16 - lo, m1)

        def gath(i):
            return pltpu.async_copy(
                cp_hbm.at[plsc.Indices(gidxb.at[i], ignored_value=-1)],
                rows[i % 3], semg[i % 3])

        def scat(i):
            return pltpu.async_copy(
                rows[i % 3],
                acc_sh.at[plsc.Indices(offb.at[i], ignored_value=-1)],
                sems[i % 3], add=True)

        B = 10

        def do_batch(b, _):
            r0 = sid * 390 + B * b
            pltpu.sync_copy(lit2d.at[pl.ds(r0, B), :], litb)
            pltpu.sync_copy(cls2d.at[pl.ds(r0, B), :], clsb)
            filter_row(0)
            filter_row(1)
            gs = {0: gath(0), 1: gath(1)}
            ss = {}
            for i in range(B):
                gs[i % 3].wait()
                ss[i % 3] = scat(i)
                if i + 2 < B:
                    filter_row(i + 2)
                    if i >= 1:
                        ss[(i + 2) % 3].wait()
                    gs[(i + 2) % 3] = gath(i + 2)
            for sl in range(3):
                ss[sl].wait()
            return 0

        lax.fori_loop(0, 390 // B, do_batch, 0)

        @pl.when(sid < EROWS % 16)
        def _():
            j = (EROWS // 16) * 16 + sid
            pltpu.sync_copy(lit2d.at[pl.ds(j, 1), :], litb.at[pl.ds(0, 1), :])
            pltpu.sync_copy(cls2d.at[pl.ds(j, 1), :], clsb.at[pl.ds(0, 1), :])
            filter_row(0)
            pltpu.sync_copy(
                cp_hbm.at[plsc.Indices(gidxb.at[0], ignored_value=-1)], rows0)
            pltpu.sync_copy(
                rows0, acc_sh.at[plsc.Indices(offb.at[0], ignored_value=-1)],
                add=True)
        plsc.subcore_barrier()

        def wb(b):
            r0 = b * 128
            pltpu.sync_copy(acc_sh.at[pl.ds(r0, 128), :], rows0)
            pltpu.sync_copy(rows0, cl_out.at[pl.ds(lo + r0, 128), :])

        def wb_round(r, _):
            wb(sid + 16 * r)
            return 0

        lax.fori_loop(0, 12, wb_round, 0)
        @pl.when(sid < 3)
        def _():
            wb(192 + sid)
        @pl.when(sid == 15)
        def _():
            pltpu.sync_copy(acc_sh.at[pl.ds(24960, 40), :],
                            rows0.at[pl.ds(0, 40), :])
            pltpu.sync_copy(rows0.at[pl.ds(0, 40), :],
                            cl_out.at[pl.ds(lo + 24960, 40), :])
        plsc.subcore_barrier()


def _k56_body(cl_ref, degf_ref, degs_ref, LiW_ref, Lib_ref, iw2_ref, ib2_ref,
              LmW1_ref, Lmb1_ref, LmW2_ref, Lmb2_ref, LvW1_ref, Lvb1_ref,
              LvW2_ref, Lvb2_ref, out_ref, sums_ref, sumsq_ref, sumdeg_ref,
              hacc_ref):
    ph = pl.program_id(0)
    p = pl.program_id(1)
    h = pl.program_id(2)
    ntot = float(2 * VARS_PER)
    w = LiW_ref[...]
    b = Lib_ref[...]
    L = cl_ref[...] + degf_ref[...] * w + b

    @pl.when(ph == 0)
    def _():
        s1 = jnp.sum(L, axis=0, keepdims=True)
        s2 = jnp.sum(L * L, axis=0, keepdims=True)
        sd = jnp.sum(degs_ref[...]) * jnp.ones((1, DIM), jnp.float32)

        @pl.when(h == 0)
        def _():
            sums_ref[pl.ds(p, 1), :] = s1
            sumsq_ref[pl.ds(p, 1), :] = s2
            sumdeg_ref[pl.ds(p, 1), :] = sd

        @pl.when(h == 1)
        def _():
            sums_ref[pl.ds(p, 1), :] += s1
            sumsq_ref[pl.ds(p, 1), :] += s2
            sumdeg_ref[pl.ds(p, 1), :] += sd

    @pl.when(ph == 1)
    def _():
        mean = sums_ref[pl.ds(p, 1), :] / ntot
        var = sumsq_ref[pl.ds(p, 1), :] / ntot - mean * mean
        std = jnp.sqrt(var + EPS)
        Ln = iw2_ref[...] * (L - mean) / std + ib2_ref[...]
        Hh = jnp.maximum(
            jnp.dot(Ln, LmW1_ref[R - 1], preferred_element_type=jnp.float32)
            + Lmb1_ref[R - 1][None], 0.0)
        hsum = jnp.sum(Hh, axis=0, keepdims=True)

        @pl.when(h == 0)
        def _():
            hacc_ref[...] = hsum

        @pl.when(h == 1)
        def _():
            Hbar = (hacc_ref[...] + hsum) / ntot
            rep = (jnp.dot(Hbar, LmW2_ref[R - 1],
                           preferred_element_type=jnp.float32)
                   + Lmb2_ref[R - 1][None]
                   + (sumdeg_ref[pl.ds(p, 1), :] / ntot) * w + b)
            z = jnp.maximum(
                jnp.dot(rep, LvW1_ref[...], preferred_element_type=jnp.float32)
                + Lvb1_ref[...], 0.0)
            out_ref[pl.ds(p, 1), :] = (
                jnp.dot(z, LvW2_ref[...], preferred_element_type=jnp.float32)
                + Lvb2_ref[...])


def _k56(cl, deg1, LiW, Lib, iw2, ib2,
         LmW1, Lmb1, LmW2, Lmb2, LvW1, Lvb1, LvW2, Lvb2):
    return pl.pallas_call(
        _k56_body,
        grid=(2, N_PROBS, 2),
        in_specs=[
            pl.BlockSpec((VARS_PER, DIM),
                         lambda ph, p, h: (p + N_PROBS * h, 0)),
            pl.BlockSpec((VARS_PER, 1),
                         lambda ph, p, h: (p + N_PROBS * (1 - h), 0)),
            pl.BlockSpec((VARS_PER, 1),
                         lambda ph, p, h: (p + N_PROBS * h, 0)),
            _full((1, DIM)), _full((1, DIM)),
            _full((1, DIM)), _full((1, DIM)),
            _full((R, DIM, DIM)), _full((R, DIM)),
            _full((R, DIM, DIM)), _full((R, DIM)),
            _full((DIM, DIM)), _full((1, DIM)),
            _full((DIM, DIM)), _full((1, DIM)),
        ],
        out_specs=_full((N_PROBS, DIM)),
        out_shape=jax.ShapeDtypeStruct((N_PROBS, DIM), jnp.float32),
        scratch_shapes=[
            pltpu.VMEM((N_PROBS, DIM), jnp.float32),
            pltpu.VMEM((N_PROBS, DIM), jnp.float32),
            pltpu.VMEM((N_PROBS, DIM), jnp.float32),
            pltpu.VMEM((1, DIM), jnp.float32),
        ],
    )(cl, deg1, deg1, LiW, Lib, iw2, ib2,
      LmW1, Lmb1, LmW2, Lmb2, LvW1, Lvb1, LvW2, Lvb2)


def kernel(lit_idx, clause_idx, L_init_W, L_init_b, C_init_W, C_init_b,
           Lp_W, Lp_b, Lm_W1, Lm_b1, Lm_W2, Lm_b2, Cm_W1, Cm_b1, Cm_W2,
           Cm_b2, Cp_W1, Cp_b1, Cp_W2, Cp_b2, in_w1, in_b1, in_w2, in_b2,
           Lv_W1, Lv_b1, Lv_W2, Lv_b2):
    lit2d = lit_idx.reshape(EROWS, 128)
    cls2d = clause_idx.reshape(EROWS, 128)

    deg_lit, deg_clause, t0, t1 = _k1(lit2d, cls2d)

    ta = t0.reshape(N_CLAUSES, 1)
    tb = t1.reshape(N_CLAUSES, 1)
    d1 = deg_clause.reshape(N_CLAUSES, 1)
    Lib = L_init_b.reshape(1, DIM)
    Cib = C_init_b.reshape(1, DIM)
    iw1 = in_w1.reshape(1, DIM)
    ib1 = in_b1.reshape(1, DIM)
    cp = _k3(ta, tb, d1, Lp_W, Lp_b, Cm_W1, Cm_b1, Cm_W2, Cm_b2,
             Cp_W1, Cp_b1, Cp_W2, Cp_b2, L_init_W, Lib, C_init_W, Cib,
             iw1, ib1)

    cl = _k4(cp, lit2d, cls2d)

    deg1 = deg_lit.reshape(N_LITS, 1)
    out = _k56(cl, deg1, L_init_W, Lib,
               in_w2.reshape(1, DIM), in_b2.reshape(1, DIM),
               Lm_W1, Lm_b1, Lm_W2, Lm_b2,
               Lv_W1, Lv_b1.reshape(1, DIM), Lv_W2, Lv_b2.reshape(1, DIM))
    return out


# speedup vs baseline: 17.2487x; 1.0269x over previous
import functools

import jax
import jax.numpy as jnp
from jax import lax
from jax.experimental import pallas as pl
from jax.experimental.pallas import tpu as pltpu
from jax.experimental.pallas import tpu_sc as plsc

N_VARS = 50000
N_LITS = 100000
N_CLAUSES = 50000
N_CELLS = 800000
N_PROBS = 10
DIM = 64
R = 4
VARS_PER = N_VARS // N_PROBS
CLAUSES_PER = N_CLAUSES // N_PROBS

EROWS = N_CELLS // 128
EPS = 1e-6

_mesh = plsc.VectorSubcoreMesh(core_axis_name="c", subcore_axis_name="s")


def _zero_vmem_1d(ref, n):
    z = jnp.zeros((16,), jnp.float32)

    def body(i, _):
        ref[pl.ds(i * 16, 16)] = z
        return 0

    lax.fori_loop(0, n // 16, body, 0)


def _fill_vmem_1d(ref, n, value):
    v = jnp.full((16,), value, jnp.float32)

    def body(i, _):
        ref[pl.ds(i * 16, 16)] = v
        return 0

    lax.fori_loop(0, n // 16, body, 0)


def _spread_1d(sid, total, bs, fn):
    full = total // bs
    tail = total - full * bs
    rounds, rem = divmod(full, 16)
    for r in range(rounds):
        fn((sid + 16 * r) * bs, bs)
    if rem:
        @pl.when(sid < rem)
        def _():
            fn((16 * rounds + sid) * bs, bs)
    if tail:
        @pl.when(sid == 15)
        def _():
            fn(full * bs, tail)


def _edge_rows(sid, fn):
    def body(k, _):
        fn(sid + 16 * k)
        return 0

    lax.fori_loop(0, EROWS // 16, body, 0)
    rem = EROWS % 16
    if rem:
        @pl.when(sid < rem)
        def _():
            fn((EROWS // 16) * 16 + sid)


@functools.partial(
    pl.kernel,
    out_type=(
        jax.ShapeDtypeStruct((N_LITS,), jnp.float32),
        jax.ShapeDtypeStruct((N_CLAUSES,), jnp.float32),
        jax.ShapeDtypeStruct((N_CLAUSES,), jnp.float32),
        jax.ShapeDtypeStruct((N_CLAUSES,), jnp.float32),
    ),
    mesh=_mesh,
    scratch_types=[
        pltpu.VMEM_SHARED((N_LITS,), jnp.float32),
        pltpu.VMEM_SHARED((N_CLAUSES,), jnp.float32),
        pltpu.VMEM_SHARED((N_CLAUSES,), jnp.float32),
        pltpu.VMEM((2048,), jnp.float32),
        pltpu.VMEM((2000,), jnp.float32),
        pltpu.VMEM((128,), jnp.float32),
        pltpu.VMEM((13, 128), jnp.int32),
        pltpu.VMEM((13, 128), jnp.int32),
        pltpu.VMEM((13, 128), jnp.float32),
        pltpu.SemaphoreType.DMA,
        pltpu.SemaphoreType.DMA,
    ],
    compiler_params=pltpu.CompilerParams(use_tc_tiling_on_sc=False),
)
def _k1(lit2d, cls2d, deg_lit_out, deg_cl_out, t0_out, t1_out,
        deg_lit_sh, deg_cl_sh, t_sh, zbuf, stage, ones_v, litb, clsb, valb,
        semA, semB):
    cid = lax.axis_index("c")
    sid = lax.axis_index("s")

    _zero_vmem_1d(zbuf, 2048)
    _fill_vmem_1d(ones_v, 128, 1.0)

    def zero_to(sh):
        def fn(off, size):
            pltpu.sync_copy(zbuf.at[pl.ds(0, size)], sh.at[pl.ds(off, size)])
        return fn

    _spread_1d(sid, N_LITS, 2000, zero_to(deg_lit_sh))
    _spread_1d(sid, N_CLAUSES, 2000, zero_to(deg_cl_sh))
    _spread_1d(sid, N_CLAUSES, 2000, zero_to(t_sh))
    plsc.subcore_barrier()

    def hist_batch(b, _):
        r0 = sid * 390 + 13 * b
        pltpu.sync_copy(lit2d.at[pl.ds(r0, 13), :], litb)
        pltpu.sync_copy(cls2d.at[pl.ds(r0, 13), :], clsb)
        ds = []
        for i in range(13):
            ds.append(pltpu.async_copy(
                ones_v, deg_lit_sh.at[litb.at[i]], semA, add=True))
            ds.append(pltpu.async_copy(
                ones_v, deg_cl_sh.at[clsb.at[i]], semB, add=True))
        for d in ds:
            d.wait()
        return 0

    lax.fori_loop(0, 30, hist_batch, 0)
    @pl.when(sid < EROWS % 16)
    def _():
        j = (EROWS // 16) * 16 + sid
        pltpu.sync_copy(lit2d.at[pl.ds(j, 1), :], litb.at[pl.ds(0, 1), :])
        pltpu.sync_copy(cls2d.at[pl.ds(j, 1), :], clsb.at[pl.ds(0, 1), :])
        pltpu.sync_copy(ones_v, deg_lit_sh.at[litb.at[0]], add=True)
        pltpu.sync_copy(ones_v, deg_cl_sh.at[clsb.at[0]], add=True)
    plsc.subcore_barrier()

    base = cid * (EROWS // 2) + sid * 195

    def t_batch(b, _):
        r0 = base + 13 * b
        pltpu.sync_copy(lit2d.at[pl.ds(r0, 13), :], litb)
        pltpu.sync_copy(cls2d.at[pl.ds(r0, 13), :], clsb)
        gs = [pltpu.async_copy(deg_lit_sh.at[litb.at[i]], valb.at[i], semA)
              for i in range(13)]
        for g in gs:
            g.wait()
        ss = [pltpu.async_copy(valb.at[i], t_sh.at[clsb.at[i]], semB, add=True)
              for i in range(13)]
        for s in ss:
            s.wait()
        return 0

    lax.fori_loop(0, 15, t_batch, 0)
    @pl.when(sid < 5)
    def _():
        j = cid * (EROWS // 2) + 3120 + sid
        pltpu.sync_copy(lit2d.at[pl.ds(j, 1), :], litb.at[pl.ds(0, 1), :])
        pltpu.sync_copy(cls2d.at[pl.ds(j, 1), :], clsb.at[pl.ds(0, 1), :])
        pltpu.sync_copy(deg_lit_sh.at[litb.at[0]], valb.at[0])
        pltpu.sync_copy(valb.at[0], t_sh.at[clsb.at[0]], add=True)
    plsc.subcore_barrier()

    def publish(sh, out):
        def fn(off, size):
            pltpu.sync_copy(sh.at[pl.ds(off, size)], stage.at[pl.ds(0, size)])
            pltpu.sync_copy(stage.at[pl.ds(0, size)], out.at[pl.ds(off, size)])
        return fn

    @pl.when(cid == 0)
    def _():
        _spread_1d(sid, N_LITS, 2000, publish(deg_lit_sh, deg_lit_out))
        _spread_1d(sid, N_CLAUSES, 2000, publish(deg_cl_sh, deg_cl_out))
        _spread_1d(sid, N_CLAUSES, 2000, publish(t_sh, t0_out))

    @pl.when(cid == 1)
    def _():
        _spread_1d(sid, N_CLAUSES, 2000, publish(t_sh, t1_out))


def _k3_body(ta_ref, tb_ref, d1_ref, LpW_ref, Lpb_ref, CmW1_ref, Cmb1_ref,
             CmW2_ref, Cmb2_ref, CpW1_ref, Cpb1_ref, CpW2_ref, Cpb2_ref,
             LiW_ref, Lib_ref, CiW_ref, Cib_ref, iw1_ref, ib1_ref, out_ref):
    n = float(CLAUSES_PER)
    t_col = ta_ref[...] + tb_ref[...]
    d_col = d1_ref[...]
    St = jnp.sum(t_col) / n
    Sd = jnp.sum(d_col) / n
    Vt = jnp.sum(t_col * t_col) / n - St * St
    Vd = jnp.sum(d_col * d_col) / n - Sd * Sd
    Ctd = jnp.sum(t_col * d_col) / n - St * Sd
    w = LiW_ref[...]
    b = Lib_ref[...]
    iw1 = iw1_ref[...]
    ib1 = ib1_ref[...]
    acc = d_col * CiW_ref[...] + Cib_ref[...]
    for i in range(R):
        Wp = LpW_ref[i]
        u = jnp.dot(w, Wp, preferred_element_type=jnp.float32)
        v = jnp.dot(b, Wp, preferred_element_type=jnp.float32) + Lpb_ref[i][None]
        var = Vt * u * u + 2.0 * Ctd * u * v + Vd * v * v
        std = jnp.sqrt(var + EPS)
        ai = iw1 * u / std
        bi = iw1 * v / std
        ei = ib1 - (St * u + Sd * v) * iw1 / std
        A = jnp.dot(ai, CmW1_ref[i], preferred_element_type=jnp.float32)
        B = jnp.dot(bi, CmW1_ref[i], preferred_element_type=jnp.float32)
        E = (jnp.dot(ei, CmW1_ref[i], preferred_element_type=jnp.float32)
             + Cmb1_ref[i][None])
        H = jnp.maximum(t_col * A + d_col * B + E, 0.0)
        acc = (acc + jnp.dot(H, CmW2_ref[i], preferred_element_type=jnp.float32)
               + Cmb2_ref[i][None])
    z = jnp.maximum(
        jnp.dot(acc, CpW1_ref[R - 1], preferred_element_type=jnp.float32)
        + Cpb1_ref[R - 1][None], 0.0)
    out_ref[...] = (jnp.dot(z, CpW2_ref[R - 1], preferred_element_type=jnp.float32)
                    + Cpb2_ref[R - 1][None])


def _full(shape):
    return pl.BlockSpec(shape, lambda *args: tuple(0 for _ in shape))


def _k3(ta, tb, d1, LpW, Lpb, CmW1, Cmb1, CmW2, Cmb2, CpW1, Cpb1, CpW2, Cpb2,
        LiW, Lib, CiW, Cib, iw1, ib1):
    return pl.pallas_call(
        _k3_body,
        grid=(N_PROBS,),
        in_specs=[
            pl.BlockSpec((CLAUSES_PER, 1), lambda p: (p, 0)),
            pl.BlockSpec((CLAUSES_PER, 1), lambda p: (p, 0)),
            pl.BlockSpec((CLAUSES_PER, 1), lambda p: (p, 0)),
            _full((R, DIM, DIM)), _full((R, DIM)),
            _full((R, DIM, DIM)), _full((R, DIM)),
            _full((R, DIM, DIM)), _full((R, DIM)),
            _full((R, DIM, DIM)), _full((R, DIM)),
            _full((R, DIM, DIM)), _full((R, DIM)),
            _full((1, DIM)), _full((1, DIM)), _full((1, DIM)), _full((1, DIM)),
            _full((1, DIM)), _full((1, DIM)),
        ],
        out_specs=pl.BlockSpec((CLAUSES_PER, DIM), lambda p: (p, 0)),
        out_shape=jax.ShapeDtypeStruct((N_CLAUSES, DIM), jnp.float32),
    )(ta, tb, d1, LpW, Lpb, CmW1, Cmb1, CmW2, Cmb2, CpW1, Cpb1, CpW2, Cpb2,
      LiW, Lib, CiW, Cib, iw1, ib1)


CHUNK_LITS = 25000
ACC_ROWS = CHUNK_LITS


@functools.partial(
    pl.kernel,
    out_type=jax.ShapeDtypeStruct((N_LITS, DIM), jnp.float32),
    mesh=_mesh,
    scratch_types=[
        pltpu.VMEM_SHARED((ACC_ROWS, DIM), jnp.float32),
        pltpu.VMEM((16, DIM), jnp.float32),
        pltpu.VMEM((10, 128), jnp.int32),
        pltpu.VMEM((10, 128), jnp.int32),
        pltpu.VMEM((10, 128), jnp.int32),
        pltpu.VMEM((10, 128), jnp.int32),
        pltpu.VMEM((128, DIM), jnp.float32),
        pltpu.VMEM((128, DIM), jnp.float32),
        pltpu.VMEM((128, DIM), jnp.float32),
        pltpu.SemaphoreType.DMA,
        pltpu.SemaphoreType.DMA,
        pltpu.SemaphoreType.DMA,
        pltpu.SemaphoreType.DMA,
        pltpu.SemaphoreType.DMA,
        pltpu.SemaphoreType.DMA,
    ],
    compiler_params=pltpu.CompilerParams(use_tc_tiling_on_sc=False),
)
def _k4(cp_hbm, lit2d, cls2d, cl_out,
        acc_sh, zrows, litb, clsb, gidxb, offb, rows0, rows1, rows2,
        semg0, semg1, semg2, sems0, sems1, sems2):
    cid = lax.axis_index("c")
    sid = lax.axis_index("s")

    def zrow_body(i, _):
        z = jnp.zeros((16,), jnp.float32)
        for q in range(DIM // 16):
            zrows[i, pl.ds(q * 16, 16)] = z
        return 0

    lax.fori_loop(0, 16, zrow_body, 0)

    rows = [rows0, rows1, rows2]
    semg = [semg0, semg1, semg2]
    sems = [sems0, sems1, sems2]

    for kchunk in range(2):
        chunk = 2 * cid + kchunk
        lo = chunk * CHUNK_LITS

        def zblk(r, _):
            pltpu.sync_copy(zrows, acc_sh.at[pl.ds((sid + 16 * r) * 16, 16), :])
            return 0

        lax.fori_loop(0, 97, zblk, 0)
        @pl.when(sid < 10)
        def _():
            pltpu.sync_copy(zrows, acc_sh.at[pl.ds((1552 + sid) * 16, 16), :])
        @pl.when(sid == 15)
        def _():
            pltpu.sync_copy(zrows.at[pl.ds(0, 8), :],
                            acc_sh.at[pl.ds(24992, 8), :])
        plsc.subcore_barrier()

        def filter_row(i):
            for q in range(8):
                l16 = litb[i, pl.ds(q * 16, 16)]
                c16 = clsb[i, pl.ds(q * 16, 16)]
                inm = (l16 >= lo) & (l16 < lo + CHUNK_LITS)
                m1 = jnp.full((16,), -1, jnp.int32)
                gidxb[i, pl.ds(q * 16, 16)] = jnp.where(inm, c16, m1)
                offb[i, pl.ds(q * 16, 16)] = jnp.where(inm, l16 - lo, m1)

        def gath(i):
            return pltpu.async_copy(
                cp_hbm.at[plsc.Indices(gidxb.at[i], ignored_value=-1)],
                rows[i % 3], semg[i % 3])

        def scat(i):
            return pltpu.async_copy(
                rows[i % 3],
                acc_sh.at[plsc.Indices(offb.at[i], ignored_value=-1)],
                sems[i % 3], add=True)

        B = 10

        def wait_slot(sl):
            pltpu.make_async_copy(
                rows[sl],
                acc_sh.at[plsc.Indices(offb.at[B - 3 + (sl - B) % 3],
                                       ignored_value=-1)],
                sems[sl]).wait()

        def do_batch(b, _):
            r0 = sid * 390 + B * b
            pltpu.sync_copy(lit2d.at[pl.ds(r0, B), :], litb)
            pltpu.sync_copy(cls2d.at[pl.ds(r0, B), :], clsb)
            filter_row(0)
            filter_row(1)

            @pl.when(b > 0)
            def _():
                for sl in range(3):
                    wait_slot(sl)

            gs = {0: gath(0), 1: gath(1)}
            ss = {}
            for i in range(B):
                gs[i % 3].wait()
                ss[i % 3] = scat(i)
                if i + 2 < B:
                    filter_row(i + 2)
                    if i >= 1:
                        ss[(i + 2) % 3].wait()
                    gs[(i + 2) % 3] = gath(i + 2)
            return 0

        lax.fori_loop(0, 390 // B, do_batch, 0)
        for sl in range(3):
            wait_slot(sl)

        @pl.when(sid < EROWS % 16)
        def _():
            j = (EROWS // 16) * 16 + sid
            pltpu.sync_copy(lit2d.at[pl.ds(j, 1), :], litb.at[pl.ds(0, 1), :])
            pltpu.sync_copy(cls2d.at[pl.ds(j, 1), :], clsb.at[pl.ds(0, 1), :])
            filter_row(0)
            pltpu.sync_copy(
                cp_hbm.at[plsc.Indices(gidxb.at[0], ignored_value=-1)], rows0)
            pltpu.sync_copy(
                rows0, acc_sh.at[plsc.Indices(offb.at[0], ignored_value=-1)],
                add=True)
        plsc.subcore_barrier()

        def wb(b):
            r0 = b * 128
            pltpu.sync_copy(acc_sh.at[pl.ds(r0, 128), :], rows0)
            pltpu.sync_copy(rows0, cl_out.at[pl.ds(lo + r0, 128), :])

        def wb_round(r, _):
            wb(sid + 16 * r)
            return 0

        lax.fori_loop(0, 12, wb_round, 0)
        @pl.when(sid < 3)
        def _():
            wb(192 + sid)
        @pl.when(sid == 15)
        def _():
            pltpu.sync_copy(acc_sh.at[pl.ds(24960, 40), :],
                            rows0.at[pl.ds(0, 40), :])
            pltpu.sync_copy(rows0.at[pl.ds(0, 40), :],
                            cl_out.at[pl.ds(lo + 24960, 40), :])
        plsc.subcore_barrier()


def _k56_body(cl_ref, degf_ref, degs_ref, LiW_ref, Lib_ref, iw2_ref, ib2_ref,
              LmW1_ref, Lmb1_ref, LmW2_ref, Lmb2_ref, LvW1_ref, Lvb1_ref,
              LvW2_ref, Lvb2_ref, out_ref, sums_ref, sumsq_ref, sumdeg_ref,
              hacc_ref):
    ph = pl.program_id(0)
    p = pl.program_id(1)
    h = pl.program_id(2)
    ntot = float(2 * VARS_PER)
    w = LiW_ref[...]
    b = Lib_ref[...]
    L = cl_ref[...] + degf_ref[...] * w + b

    @pl.when(ph == 0)
    def _():
        s1 = jnp.sum(L, axis=0, keepdims=True)
        s2 = jnp.sum(L * L, axis=0, keepdims=True)
        sd = jnp.sum(degs_ref[...]) * jnp.ones((1, DIM), jnp.float32)

        @pl.when(h == 0)
        def _():
            sums_ref[pl.ds(p, 1), :] = s1
            sumsq_ref[pl.ds(p, 1), :] = s2
            sumdeg_ref[pl.ds(p, 1), :] = sd

        @pl.when(h == 1)
        def _():
            sums_ref[pl.ds(p, 1), :] += s1
            sumsq_ref[pl.ds(p, 1), :] += s2
            sumdeg_ref[pl.ds(p, 1), :] += sd

    @pl.when(ph == 1)
    def _():
        mean = sums_ref[pl.ds(p, 1), :] / ntot
        var = sumsq_ref[pl.ds(p, 1), :] / ntot - mean * mean
        std = jnp.sqrt(var + EPS)
        Ln = iw2_ref[...] * (L - mean) / std + ib2_ref[...]
        Hh = jnp.maximum(
            jnp.dot(Ln, LmW1_ref[R - 1], preferred_element_type=jnp.float32)
            + Lmb1_ref[R - 1][None], 0.0)
        hsum = jnp.sum(Hh, axis=0, keepdims=True)

        @pl.when(h == 0)
        def _():
            hacc_ref[...] = hsum

        @pl.when(h == 1)
        def _():
            Hbar = (hacc_ref[...] + hsum) / ntot
            rep = (jnp.dot(Hbar, LmW2_ref[R - 1],
                           preferred_element_type=jnp.float32)
                   + Lmb2_ref[R - 1][None]
                   + (sumdeg_ref[pl.ds(p, 1), :] / ntot) * w + b)
            z = jnp.maximum(
                jnp.dot(rep, LvW1_ref[...], preferred_element_type=jnp.float32)
                + Lvb1_ref[...], 0.0)
            out_ref[pl.ds(p, 1), :] = (
                jnp.dot(z, LvW2_ref[...], preferred_element_type=jnp.float32)
                + Lvb2_ref[...])


def _k56(cl, deg1, LiW, Lib, iw2, ib2,
         LmW1, Lmb1, LmW2, Lmb2, LvW1, Lvb1, LvW2, Lvb2):
    return pl.pallas_call(
        _k56_body,
        grid=(2, N_PROBS, 2),
        in_specs=[
            pl.BlockSpec((VARS_PER, DIM),
                         lambda ph, p, h: (p + N_PROBS * h, 0)),
            pl.BlockSpec((VARS_PER, 1),
                         lambda ph, p, h: (p + N_PROBS * (1 - h), 0)),
            pl.BlockSpec((VARS_PER, 1),
                         lambda ph, p, h: (p + N_PROBS * h, 0)),
            _full((1, DIM)), _full((1, DIM)),
            _full((1, DIM)), _full((1, DIM)),
            _full((R, DIM, DIM)), _full((R, DIM)),
            _full((R, DIM, DIM)), _full((R, DIM)),
            _full((DIM, DIM)), _full((1, DIM)),
            _full((DIM, DIM)), _full((1, DIM)),
        ],
        out_specs=_full((N_PROBS, DIM)),
        out_shape=jax.ShapeDtypeStruct((N_PROBS, DIM), jnp.float32),
        scratch_shapes=[
            pltpu.VMEM((N_PROBS, DIM), jnp.float32),
            pltpu.VMEM((N_PROBS, DIM), jnp.float32),
            pltpu.VMEM((N_PROBS, DIM), jnp.float32),
            pltpu.VMEM((1, DIM), jnp.float32),
        ],
    )(cl, deg1, deg1, LiW, Lib, iw2, ib2,
      LmW1, Lmb1, LmW2, Lmb2, LvW1, Lvb1, LvW2, Lvb2)


def kernel(lit_idx, clause_idx, L_init_W, L_init_b, C_init_W, C_init_b,
           Lp_W, Lp_b, Lm_W1, Lm_b1, Lm_W2, Lm_b2, Cm_W1, Cm_b1, Cm_W2,
           Cm_b2, Cp_W1, Cp_b1, Cp_W2, Cp_b2, in_w1, in_b1, in_w2, in_b2,
           Lv_W1, Lv_b1, Lv_W2, Lv_b2):
    lit2d = lit_idx.reshape(EROWS, 128)
    cls2d = clause_idx.reshape(EROWS, 128)

    deg_lit, deg_clause, t0, t1 = _k1(lit2d, cls2d)

    ta = t0.reshape(N_CLAUSES, 1)
    tb = t1.reshape(N_CLAUSES, 1)
    d1 = deg_clause.reshape(N_CLAUSES, 1)
    Lib = L_init_b.reshape(1, DIM)
    Cib = C_init_b.reshape(1, DIM)
    iw1 = in_w1.reshape(1, DIM)
    ib1 = in_b1.reshape(1, DIM)
    cp = _k3(ta, tb, d1, Lp_W, Lp_b, Cm_W1, Cm_b1, Cm_W2, Cm_b2,
             Cp_W1, Cp_b1, Cp_W2, Cp_b2, L_init_W, Lib, C_init_W, Cib,
             iw1, ib1)

    cl = _k4(cp, lit2d, cls2d)

    deg1 = deg_lit.reshape(N_LITS, 1)
    out = _k56(cl, deg1, L_init_W, Lib,
               in_w2.reshape(1, DIM), in_b2.reshape(1, DIM),
               Lm_W1, Lm_b1, Lm_W2, Lm_b2,
               Lv_W1, Lv_b1.reshape(1, DIM), Lv_W2, Lv_b2.reshape(1, DIM))
    return out
